# Initial kernel scaffold; baseline (speedup 1.0000x reference)
import jax
import jax.numpy as jnp
from jax.experimental import pallas as pl


def kernel(x, edge_index, edge_rating, rating_W, sage_W):
    raise NotImplementedError("write your pallas kernel here")


# speedup vs baseline: 3.7233x; 3.7233x over previous
import functools

import jax
import jax.numpy as jnp
from jax import lax
from jax.experimental import pallas as pl
from jax.experimental.pallas import tpu as pltpu
from jax.experimental.pallas import tpu_sc as plsc

R = 10
NEG = 0.1
N = 10000
E = 320000
D_IN = 128
D_H = 256
D_OUT = 256
NK = N * R

NC, NS = 2, 16
NW = NC * NS

EPT_A = E // NW
CHA = 2000

CK = 6400
NKP = 102400
NCH = NKP // CK
SLICE = CK // NS
ZR = 40
EPT_C = E // NS
CHC = 2000
BATCH = 128
MAXM = CHC + BATCH + 48


def _mesh():
    return plsc.VectorSubcoreMesh(
        core_axis_name="c", subcore_axis_name="s", num_cores=NC, num_subcores=NS
    )


def _degrees_and_keys(dst, src, edge_rating, zeros_nk):
    @functools.partial(
        pl.kernel,
        out_type=(
            jax.ShapeDtypeStruct((E,), jnp.int32),
            jax.ShapeDtypeStruct((E,), jnp.int32),
            jax.ShapeDtypeStruct((NW * NK,), jnp.float32),
            jax.ShapeDtypeStruct((NW * NK,), jnp.float32),
        ),
        mesh=_mesh(),
        compiler_params=pltpu.CompilerParams(needs_layout_passes=False),
        scratch_types=[
            pltpu.VMEM((CHA,), jnp.int32),
            pltpu.VMEM((CHA,), jnp.int32),
            pltpu.VMEM((CHA,), jnp.int32),
            pltpu.VMEM((NK,), jnp.float32),
        ],
    )
    def k(dst_hbm, src_hbm, er_hbm, z_hbm, kd_hbm, ks_hbm, degd_hbm, degs_hbm,
          nbuf, rbuf, kbuf, table):
        wid = lax.axis_index("s") * NC + lax.axis_index("c")
        base = wid * EPT_A
        ones = jnp.ones((16,), jnp.float32)
        for side in range(2):
            nodes_hbm = dst_hbm if side == 0 else src_hbm
            keys_hbm = kd_hbm if side == 0 else ks_hbm
            deg_hbm = degd_hbm if side == 0 else degs_hbm
            pltpu.sync_copy(z_hbm, table)

            def chunk_body(cc, _):
                eb = base + cc * CHA
                pltpu.sync_copy(nodes_hbm.at[pl.ds(eb, CHA)], nbuf)
                pltpu.sync_copy(er_hbm.at[pl.ds(eb, CHA)], rbuf)

                def inner(i, _):
                    nd = nbuf[pl.ds(i * 16, 16)]
                    rt = rbuf[pl.ds(i * 16, 16)]
                    kk = nd * R + rt
                    kbuf[pl.ds(i * 16, 16)] = kk
                    plsc.addupdate_scatter(table, [kk], ones)
                    return 0

                lax.fori_loop(0, CHA // 16, inner, 0)
                pltpu.sync_copy(kbuf, keys_hbm.at[pl.ds(eb, CHA)])
                return 0

            lax.fori_loop(0, EPT_A // CHA, chunk_body, 0)
            pltpu.sync_copy(table, deg_hbm.at[pl.ds(wid * NK, NK)])

    return k(dst, src, edge_rating, zeros_nk)


def _scale_body(x_ref, degs_ref, degd_ref, xs_ref, invd_ref):
    ds_ = jnp.sum(degs_ref[...], axis=0)
    inv_s = lax.rsqrt(jnp.maximum(ds_, 1.0))
    xs_ref[...] = x_ref[...][:, None, :] * inv_s[:, :, None]
    dd_ = jnp.sum(degd_ref[...], axis=0)
    invd_ref[...] = lax.rsqrt(jnp.maximum(dd_, 1.0))


def _build_scaled(x, deg_s3, deg_d3):
    BN = 400
    return pl.pallas_call(
        _scale_body,
        grid=(N // BN,),
        in_specs=[
            pl.BlockSpec((BN, D_IN), lambda i: (i, 0)),
            pl.BlockSpec((NW, BN, R), lambda i: (0, i, 0)),
            pl.BlockSpec((NW, BN, R), lambda i: (0, i, 0)),
        ],
        out_specs=[
            pl.BlockSpec((BN, R, D_IN), lambda i: (i, 0, 0)),
            pl.BlockSpec((BN, R), lambda i: (i, 0)),
        ],
        out_shape=[
            jax.ShapeDtypeStruct((N, R, D_IN), jnp.float32),
            jax.ShapeDtypeStruct((N, R), jnp.float32),
        ],
    )(x, deg_s3, deg_d3)


def _aggregate(kd, ks, xs):
    @functools.partial(
        pl.kernel,
        out_type=jax.ShapeDtypeStruct((NKP, D_IN), jnp.float32),
        mesh=_mesh(),
        compiler_params=pltpu.CompilerParams(needs_layout_passes=False),
        scratch_types=[
            pltpu.VMEM((CHC,), jnp.int32),
            pltpu.VMEM((CHC,), jnp.int32),
            pltpu.VMEM((MAXM,), jnp.int32),
            pltpu.VMEM((MAXM,), jnp.int32),
            pltpu.VMEM((BATCH,), jnp.int32),
            pltpu.VMEM((BATCH,), jnp.int32),
            pltpu.VMEM((BATCH, D_IN), jnp.float32),
            pltpu.VMEM((BATCH,), jnp.int32),
            pltpu.VMEM((BATCH,), jnp.int32),
            pltpu.VMEM((BATCH, D_IN), jnp.float32),
            pltpu.VMEM((ZR, D_IN), jnp.float32),
            pltpu.VMEM_SHARED((CK + 8, D_IN), jnp.float32),
            pltpu.SemaphoreType.DMA,
        ],
    )
    def k(kd_hbm, ks_hbm, xs_hbm, pre_hbm, kdb, ksb, didx, gidx,
          d0, g0, r0, d1, g1, r1, zbuf, acc, ssem):
        cid = lax.axis_index("c")
        sid = lax.axis_index("s")
        ebase = sid * EPT_C
        zero16 = jnp.zeros((16,), jnp.float32)

        def zb(i, _):
            row = i // (D_IN // 16)
            col = i % (D_IN // 16)
            zbuf[row, pl.ds(col * 16, 16)] = zero16
            return 0

        lax.fori_loop(0, ZR * (D_IN // 16), zb, 0)

        def fire_one(b, f):
            def do(dd, gg, rb):
                @pl.when(f >= 2)
                def _():
                    pltpu.make_async_copy(rb, acc.at[dd], ssem).wait()

                def cp(kk, _):
                    dd[pl.ds(kk * 16, 16)] = didx[pl.ds(b * BATCH + kk * 16, 16)]
                    gg[pl.ds(kk * 16, 16)] = gidx[pl.ds(b * BATCH + kk * 16, 16)]
                    return 0

                lax.fori_loop(0, BATCH // 16, cp, 0)
                pltpu.sync_copy(xs_hbm.at[gg], rb)
                pltpu.async_copy(rb, acc.at[dd], ssem, add=True)

            @pl.when(f % 2 == 0)
            def _():
                do(d0, g0, r0)

            @pl.when(f % 2 == 1)
            def _():
                do(d1, g1, r1)

        def chunk_body(cc, _):
            c = cc * NC + cid
            lo = c * CK

            def zc(z, _):
                pltpu.sync_copy(zbuf, acc.at[pl.ds(sid * SLICE + z * ZR, ZR)])
                return 0

            lax.fori_loop(0, SLICE // ZR, zc, 0)
            plsc.subcore_barrier()

            def scan_blk(blk, carry):
                rem, fires = carry
                eb = ebase + blk * CHC
                pltpu.sync_copy(kd_hbm.at[pl.ds(eb, CHC)], kdb)
                pltpu.sync_copy(ks_hbm.at[pl.ds(eb, CHC)], ksb)

                def sc_in(i, cnt):
                    kdv = kdb[pl.ds(i * 16, 16)]
                    ksv = ksb[pl.ds(i * 16, 16)]
                    m = (kdv >= lo) & (kdv < lo + CK)
                    plsc.store_compressed(didx.at[pl.ds(cnt, 16)], kdv - lo, mask=m)
                    plsc.store_compressed(gidx.at[pl.ds(cnt, 16)], ksv, mask=m)
                    return cnt + jnp.sum(m.astype(jnp.int32))

                cnt = lax.fori_loop(0, CHC // 16, sc_in, rem)
                nfull = cnt // BATCH

                def fb(b, f):
                    fire_one(b, f)
                    return f + 1

                fires = lax.fori_loop(0, nfull, fb, fires)
                newrem = cnt - nfull * BATCH

                def mv(kk, _):
                    @pl.when(kk * 16 < newrem)
                    def _():
                        didx[pl.ds(kk * 16, 16)] = didx[
                            pl.ds(nfull * BATCH + kk * 16, 16)
                        ]
                        gidx[pl.ds(kk * 16, 16)] = gidx[
                            pl.ds(nfull * BATCH + kk * 16, 16)
                        ]

                    return 0

                lax.fori_loop(0, BATCH // 16, mv, 0)
                return (newrem, fires)

            rem, fires = lax.fori_loop(
                0, EPT_C // CHC, scan_blk, (jnp.int32(0), jnp.int32(0))
            )

            @pl.when(rem > 0)
            def _():
                def padb(j, _):
                    off = rem + j * 16

                    @pl.when(off < BATCH)
                    def _():
                        didx[pl.ds(off, 16)] = jnp.full((16,), CK, jnp.int32)
                        gidx[pl.ds(off, 16)] = jnp.zeros((16,), jnp.int32)

                    return 0

                lax.fori_loop(0, BATCH // 16, padb, 0)
                fire_one(0, fires)

            total = fires + (rem > 0).astype(jnp.int32)

            @pl.when(total >= 1)
            def _():
                pltpu.make_async_copy(r0, acc.at[d0], ssem).wait()

            @pl.when(total >= 2)
            def _():
                pltpu.make_async_copy(r0, acc.at[d0], ssem).wait()

            plsc.subcore_barrier()

            pltpu.sync_copy(
                acc.at[pl.ds(sid * SLICE, SLICE)],
                pre_hbm.at[pl.ds(lo + sid * SLICE, SLICE)],
            )
            return 0

        lax.fori_loop(0, NCH // NC, chunk_body, 0)

    return k(kd, ks, xs)


def _output_body(pre_ref, invd_ref, w3_ref, sw_ref, out_ref):
    p = pre_ref[...] * invd_ref[...][:, :, None]
    acc = jnp.dot(p[:, 0, :], w3_ref[0], preferred_element_type=jnp.float32)
    for r in range(1, R):
        acc += jnp.dot(p[:, r, :], w3_ref[r], preferred_element_type=jnp.float32)
    comb = jnp.where(acc >= 0, acc, NEG * acc)
    o = jnp.dot(comb, sw_ref[...], preferred_element_type=jnp.float32)
    out_ref[...] = jnp.where(o >= 0, o, NEG * o)


def _dense_out(pre3, inv_d3, w3, sage_wt):
    BN = 400
    return pl.pallas_call(
        _output_body,
        grid=(N // BN,),
        in_specs=[
            pl.BlockSpec((BN, R, D_IN), lambda i: (i, 0, 0)),
            pl.BlockSpec((BN, R), lambda i: (i, 0)),
            pl.BlockSpec((R, D_IN, D_H), lambda i: (0, 0, 0)),
            pl.BlockSpec((D_H, D_OUT), lambda i: (0, 0)),
        ],
        out_specs=pl.BlockSpec((BN, D_OUT), lambda i: (i, 0)),
        out_shape=jax.ShapeDtypeStruct((N, D_OUT), jnp.float32),
    )(pre3, inv_d3, w3, sage_wt)


def kernel(x, edge_index, edge_rating, rating_W, sage_W):
    w3 = jnp.transpose(rating_W, (0, 2, 1))
    sage_wt = sage_W.T
    zeros_nk = jnp.zeros((NK,), jnp.float32)

    kd, ks, deg_d, deg_s = _degrees_and_keys(
        edge_index[0], edge_index[1], edge_rating, zeros_nk
    )
    deg_d3 = deg_d.reshape(NW, N, R)
    deg_s3 = deg_s.reshape(NW, N, R)
    xs3, inv_d3 = _build_scaled(x, deg_s3, deg_d3)
    pre = _aggregate(kd, ks, xs3.reshape(NK, D_IN))
    pre3 = pre.reshape(NKP // R, R, D_IN)
    return _dense_out(pre3, inv_d3, w3, sage_wt)


# speedup vs baseline: 4.7728x; 1.2819x over previous
import functools

import jax
import jax.numpy as jnp
from jax import lax
from jax.experimental import pallas as pl
from jax.experimental.pallas import tpu as pltpu
from jax.experimental.pallas import tpu_sc as plsc

R = 10
NEG = 0.1
N = 10000
E = 320000
D_IN = 128
D_H = 256
D_OUT = 256
NK = N * R

NC, NS = 2, 16
NW = NC * NS

EPT_A = E // NW
CHA = 2000

CK = 10240
NKP = 102400
NCH = NKP // CK
SLICE = CK // NS
ZR = 40
EPT_C = E // NS
CHC = 2000
BATCH = 128
MAXM = CHC + BATCH + 48


def _mesh():
    return plsc.VectorSubcoreMesh(
        core_axis_name="c", subcore_axis_name="s", num_cores=NC, num_subcores=NS
    )


def _degrees_and_keys(dst, src, edge_rating, zeros_nk):
    @functools.partial(
        pl.kernel,
        out_type=(
            jax.ShapeDtypeStruct((E,), jnp.int32),
            jax.ShapeDtypeStruct((E,), jnp.int32),
            jax.ShapeDtypeStruct((NW * NK,), jnp.float32),
            jax.ShapeDtypeStruct((NW * NK,), jnp.float32),
        ),
        mesh=_mesh(),
        compiler_params=pltpu.CompilerParams(needs_layout_passes=False),
        scratch_types=[
            pltpu.VMEM((CHA,), jnp.int32),
            pltpu.VMEM((CHA,), jnp.int32),
            pltpu.VMEM((CHA,), jnp.int32),
            pltpu.VMEM((NK,), jnp.float32),
        ],
    )
    def k(dst_hbm, src_hbm, er_hbm, z_hbm, kd_hbm, ks_hbm, degd_hbm, degs_hbm,
          nbuf, rbuf, kbuf, table):
        wid = lax.axis_index("s") * NC + lax.axis_index("c")
        base = wid * EPT_A
        ones = jnp.ones((16,), jnp.float32)
        for side in range(2):
            nodes_hbm = dst_hbm if side == 0 else src_hbm
            keys_hbm = kd_hbm if side == 0 else ks_hbm
            deg_hbm = degd_hbm if side == 0 else degs_hbm
            pltpu.sync_copy(z_hbm, table)

            def chunk_body(cc, _):
                eb = base + cc * CHA
                pltpu.sync_copy(nodes_hbm.at[pl.ds(eb, CHA)], nbuf)
                pltpu.sync_copy(er_hbm.at[pl.ds(eb, CHA)], rbuf)

                def inner(i, _):
                    nd = nbuf[pl.ds(i * 16, 16)]
                    rt = rbuf[pl.ds(i * 16, 16)]
                    kk = nd * R + rt
                    kbuf[pl.ds(i * 16, 16)] = kk
                    plsc.addupdate_scatter(table, [kk], ones)
                    return 0

                lax.fori_loop(0, CHA // 16, inner, 0)
                pltpu.sync_copy(kbuf, keys_hbm.at[pl.ds(eb, CHA)])
                return 0

            lax.fori_loop(0, EPT_A // CHA, chunk_body, 0)
            pltpu.sync_copy(table, deg_hbm.at[pl.ds(wid * NK, NK)])

    return k(dst, src, edge_rating, zeros_nk)


def _scale_body(x_ref, degs_ref, degd_ref, xs_ref, invd_ref):
    ds_ = jnp.sum(degs_ref[...], axis=0)
    inv_s = lax.rsqrt(jnp.maximum(ds_, 1.0))
    xs_ref[...] = x_ref[...][:, None, :] * inv_s[:, :, None]
    dd_ = jnp.sum(degd_ref[...], axis=0)
    invd_ref[...] = lax.rsqrt(jnp.maximum(dd_, 1.0))


def _build_scaled(x, deg_s3, deg_d3):
    BN = 400
    return pl.pallas_call(
        _scale_body,
        grid=(N // BN,),
        in_specs=[
            pl.BlockSpec((BN, D_IN), lambda i: (i, 0)),
            pl.BlockSpec((NW, BN, R), lambda i: (0, i, 0)),
            pl.BlockSpec((NW, BN, R), lambda i: (0, i, 0)),
        ],
        out_specs=[
            pl.BlockSpec((BN, R, D_IN), lambda i: (i, 0, 0)),
            pl.BlockSpec((BN, R), lambda i: (i, 0)),
        ],
        out_shape=[
            jax.ShapeDtypeStruct((N, R, D_IN), jnp.float32),
            jax.ShapeDtypeStruct((N, R), jnp.float32),
        ],
    )(x, deg_s3, deg_d3)


def _aggregate(kd, ks, xs):
    @functools.partial(
        pl.kernel,
        out_type=jax.ShapeDtypeStruct((NKP, D_IN), jnp.float32),
        mesh=_mesh(),
        compiler_params=pltpu.CompilerParams(needs_layout_passes=False),
        scratch_types=[
            pltpu.VMEM((CHC,), jnp.int32),
            pltpu.VMEM((CHC,), jnp.int32),
            pltpu.VMEM((MAXM,), jnp.int32),
            pltpu.VMEM((MAXM,), jnp.int32),
            pltpu.VMEM((BATCH,), jnp.int32),
            pltpu.VMEM((BATCH,), jnp.int32),
            pltpu.VMEM((BATCH, D_IN), jnp.float32),
            pltpu.VMEM((BATCH,), jnp.int32),
            pltpu.VMEM((BATCH,), jnp.int32),
            pltpu.VMEM((BATCH, D_IN), jnp.float32),
            pltpu.VMEM((ZR, D_IN), jnp.float32),
            pltpu.VMEM_SHARED((CK + 8, D_IN), jnp.float32),
            pltpu.SemaphoreType.DMA,
            pltpu.SemaphoreType.DMA,
        ],
    )
    def k(kd_hbm, ks_hbm, xs_hbm, pre_hbm, kdb, ksb, didx, gidx,
          d0, g0, r0, d1, g1, r1, zbuf, acc, ssem, gsem):
        cid = lax.axis_index("c")
        sid = lax.axis_index("s")
        ebase = sid * EPT_C
        zero16 = jnp.zeros((16,), jnp.float32)

        def zb(i, _):
            row = i // (D_IN // 16)
            col = i % (D_IN // 16)
            zbuf[row, pl.ds(col * 16, 16)] = zero16
            return 0

        lax.fori_loop(0, ZR * (D_IN // 16), zb, 0)

        def fire_one(b, f):
            def do(dd, gg, rb, od, og, orb):
                @pl.when(f >= 2)
                def _():
                    pltpu.make_async_copy(rb, acc.at[dd], ssem).wait()

                def cp(kk, _):
                    dd[pl.ds(kk * 16, 16)] = didx[pl.ds(b * BATCH + kk * 16, 16)]
                    gg[pl.ds(kk * 16, 16)] = gidx[pl.ds(b * BATCH + kk * 16, 16)]
                    return 0

                lax.fori_loop(0, BATCH // 16, cp, 0)
                pltpu.async_copy(xs_hbm.at[gg], rb, gsem)

                @pl.when(f >= 1)
                def _():
                    pltpu.make_async_copy(xs_hbm.at[og], orb, gsem).wait()
                    pltpu.async_copy(orb, acc.at[od], ssem, add=True)

            @pl.when(f % 2 == 0)
            def _():
                do(d0, g0, r0, d1, g1, r1)

            @pl.when(f % 2 == 1)
            def _():
                do(d1, g1, r1, d0, g0, r0)

        def chunk_body(cc, _):
            c = cc * NC + cid
            lo = c * CK

            def zc(z, _):
                pltpu.sync_copy(zbuf, acc.at[pl.ds(sid * SLICE + z * ZR, ZR)])
                return 0

            lax.fori_loop(0, SLICE // ZR, zc, 0)
            plsc.subcore_barrier()

            def scan_blk(blk, carry):
                rem, fires = carry
                eb = ebase + blk * CHC
                pltpu.sync_copy(kd_hbm.at[pl.ds(eb, CHC)], kdb)
                pltpu.sync_copy(ks_hbm.at[pl.ds(eb, CHC)], ksb)

                def sc_in(i, cnt):
                    kdv = kdb[pl.ds(i * 16, 16)]
                    ksv = ksb[pl.ds(i * 16, 16)]
                    m = (kdv >= lo) & (kdv < lo + CK)
                    plsc.store_compressed(didx.at[pl.ds(cnt, 16)], kdv - lo, mask=m)
                    plsc.store_compressed(gidx.at[pl.ds(cnt, 16)], ksv, mask=m)
                    return cnt + plsc.all_reduce_population_count(m)[0]

                cnt = lax.fori_loop(0, CHC // 16, sc_in, rem)
                nfull = cnt // BATCH

                def fb(b, f):
                    fire_one(b, f)
                    return f + 1

                fires = lax.fori_loop(0, nfull, fb, fires)
                newrem = cnt - nfull * BATCH

                def mv(kk, _):
                    @pl.when(kk * 16 < newrem)
                    def _():
                        didx[pl.ds(kk * 16, 16)] = didx[
                            pl.ds(nfull * BATCH + kk * 16, 16)
                        ]
                        gidx[pl.ds(kk * 16, 16)] = gidx[
                            pl.ds(nfull * BATCH + kk * 16, 16)
                        ]

                    return 0

                lax.fori_loop(0, BATCH // 16, mv, 0)
                return (newrem, fires)

            rem, fires = lax.fori_loop(
                0, EPT_C // CHC, scan_blk, (jnp.int32(0), jnp.int32(0))
            )

            @pl.when(rem > 0)
            def _():
                def padb(j, _):
                    off = rem + j * 16

                    @pl.when(off < BATCH)
                    def _():
                        didx[pl.ds(off, 16)] = jnp.full((16,), CK, jnp.int32)
                        gidx[pl.ds(off, 16)] = jnp.zeros((16,), jnp.int32)

                    return 0

                lax.fori_loop(0, BATCH // 16, padb, 0)
                fire_one(0, fires)

            total = fires + (rem > 0).astype(jnp.int32)

            @pl.when(total >= 1)
            def _():
                last = (total - 1) % 2

                @pl.when(last == 0)
                def _():
                    pltpu.make_async_copy(xs_hbm.at[g0], r0, gsem).wait()
                    pltpu.async_copy(r0, acc.at[d0], ssem, add=True)

                @pl.when(last == 1)
                def _():
                    pltpu.make_async_copy(xs_hbm.at[g1], r1, gsem).wait()
                    pltpu.async_copy(r1, acc.at[d1], ssem, add=True)

            @pl.when(total >= 1)
            def _():
                pltpu.make_async_copy(r0, acc.at[d0], ssem).wait()

            @pl.when(total >= 2)
            def _():
                pltpu.make_async_copy(r0, acc.at[d0], ssem).wait()

            plsc.subcore_barrier()

            pltpu.sync_copy(
                acc.at[pl.ds(sid * SLICE, SLICE)],
                pre_hbm.at[pl.ds(lo + sid * SLICE, SLICE)],
            )
            return 0

        lax.fori_loop(0, NCH // NC, chunk_body, 0)

    return k(kd, ks, xs)


def _output_body(pre_ref, invd_ref, w3_ref, sw_ref, out_ref):
    p = pre_ref[...] * invd_ref[...][:, :, None]
    acc = jnp.dot(p[:, 0, :], w3_ref[0], preferred_element_type=jnp.float32)
    for r in range(1, R):
        acc += jnp.dot(p[:, r, :], w3_ref[r], preferred_element_type=jnp.float32)
    comb = jnp.where(acc >= 0, acc, NEG * acc)
    o = jnp.dot(comb, sw_ref[...], preferred_element_type=jnp.float32)
    out_ref[...] = jnp.where(o >= 0, o, NEG * o)


def _dense_out(pre3, inv_d3, w3, sage_wt):
    BN = 400
    return pl.pallas_call(
        _output_body,
        grid=(N // BN,),
        in_specs=[
            pl.BlockSpec((BN, R, D_IN), lambda i: (i, 0, 0)),
            pl.BlockSpec((BN, R), lambda i: (i, 0)),
            pl.BlockSpec((R, D_IN, D_H), lambda i: (0, 0, 0)),
            pl.BlockSpec((D_H, D_OUT), lambda i: (0, 0)),
        ],
        out_specs=pl.BlockSpec((BN, D_OUT), lambda i: (i, 0)),
        out_shape=jax.ShapeDtypeStruct((N, D_OUT), jnp.float32),
    )(pre3, inv_d3, w3, sage_wt)


def kernel(x, edge_index, edge_rating, rating_W, sage_W):
    w3 = jnp.transpose(rating_W, (0, 2, 1))
    sage_wt = sage_W.T
    zeros_nk = jnp.zeros((NK,), jnp.float32)

    kd, ks, deg_d, deg_s = _degrees_and_keys(
        edge_index[0], edge_index[1], edge_rating, zeros_nk
    )
    deg_d3 = deg_d.reshape(NW, N, R)
    deg_s3 = deg_s.reshape(NW, N, R)
    xs3, inv_d3 = _build_scaled(x, deg_s3, deg_d3)
    pre = _aggregate(kd, ks, xs3.reshape(NK, D_IN))
    pre3 = pre.reshape(NKP // R, R, D_IN)
    return _dense_out(pre3, inv_d3, w3, sage_wt)


# speedup vs baseline: 8.2495x; 1.7284x over previous
import functools

import jax
import jax.numpy as jnp
from jax import lax
from jax.experimental import pallas as pl
from jax.experimental.pallas import tpu as pltpu
from jax.experimental.pallas import tpu_sc as plsc

R = 10
NEG = 0.1
N = 10000
NP = 10240
E = 320000
D_IN = 128
D_H = 256
D_OUT = 256
NK = R * NP

NC, NS = 2, 16
NW = NC * NS

EPT_A = E // NW
CHA = 2000

CK = 10240
NCH = NK // CK
SLICE = CK // NS
ZR = 40
EPT_C = E // NS
CHC = 2000
BATCH = 128
MAXM = CHC + BATCH + 48


def _mesh():
    return plsc.VectorSubcoreMesh(
        core_axis_name="c", subcore_axis_name="s", num_cores=NC, num_subcores=NS
    )


def _degrees_and_keys(dst, src, edge_rating, zeros_nk):
    @functools.partial(
        pl.kernel,
        out_type=(
            jax.ShapeDtypeStruct((E,), jnp.int32),
            jax.ShapeDtypeStruct((E,), jnp.int32),
            jax.ShapeDtypeStruct((NW * NK,), jnp.float32),
            jax.ShapeDtypeStruct((NW * NK,), jnp.float32),
        ),
        mesh=_mesh(),
        compiler_params=pltpu.CompilerParams(needs_layout_passes=False),
        scratch_types=[
            pltpu.VMEM((CHA,), jnp.int32),
            pltpu.VMEM((CHA,), jnp.int32),
            pltpu.VMEM((CHA,), jnp.int32),
            pltpu.VMEM((NK,), jnp.float32),
        ],
    )
    def k(dst_hbm, src_hbm, er_hbm, z_hbm, kd_hbm, ks_hbm, degd_hbm, degs_hbm,
          nbuf, rbuf, kbuf, table):
        wid = lax.axis_index("s") * NC + lax.axis_index("c")
        base = wid * EPT_A
        ones = jnp.ones((16,), jnp.float32)
        for side in range(2):
            nodes_hbm = dst_hbm if side == 0 else src_hbm
            keys_hbm = kd_hbm if side == 0 else ks_hbm
            deg_hbm = degd_hbm if side == 0 else degs_hbm
            pltpu.sync_copy(z_hbm, table)

            def chunk_body(cc, _):
                eb = base + cc * CHA
                pltpu.sync_copy(nodes_hbm.at[pl.ds(eb, CHA)], nbuf)
                pltpu.sync_copy(er_hbm.at[pl.ds(eb, CHA)], rbuf)

                def inner(i, _):
                    nd = nbuf[pl.ds(i * 16, 16)]
                    rt = rbuf[pl.ds(i * 16, 16)]
                    kk = rt * NP + nd
                    kbuf[pl.ds(i * 16, 16)] = kk
                    plsc.addupdate_scatter(table, [kk], ones)
                    return 0

                lax.fori_loop(0, CHA // 16, inner, 0)
                pltpu.sync_copy(kbuf, keys_hbm.at[pl.ds(eb, CHA)])
                return 0

            lax.fori_loop(0, EPT_A // CHA, chunk_body, 0)
            pltpu.sync_copy(table, deg_hbm.at[pl.ds(wid * NK, NK)])

    return k(dst, src, edge_rating, zeros_nk)


def _reduce_body(degd_ref, degs_ref, outd_ref, outs_ref):
    outd_ref[...] = jnp.sum(degd_ref[...], axis=0)
    outs_ref[...] = jnp.sum(degs_ref[...], axis=0)


def _reduce_partials(deg_d2, deg_s2):
    CB = 4096
    return pl.pallas_call(
        _reduce_body,
        grid=(NK // CB,),
        in_specs=[
            pl.BlockSpec((NW, CB), lambda i: (0, i)),
            pl.BlockSpec((NW, CB), lambda i: (0, i)),
        ],
        out_specs=[
            pl.BlockSpec((CB,), lambda i: (i,)),
            pl.BlockSpec((CB,), lambda i: (i,)),
        ],
        out_shape=[
            jax.ShapeDtypeStruct((NK,), jnp.float32),
            jax.ShapeDtypeStruct((NK,), jnp.float32),
        ],
    )(deg_d2, deg_s2)


def _scale_body(x_ref, degs_ref, degd_ref, xs_ref, invd_ref):
    inv_s = lax.rsqrt(jnp.maximum(degs_ref[...], 1.0))
    xs_ref[...] = x_ref[...][None, :, :] * inv_s[:, :, None]
    invd_ref[...] = lax.rsqrt(jnp.maximum(degd_ref[...], 1.0))


def _build_scaled(x, deg_s2, deg_d2):
    BN = 512
    return pl.pallas_call(
        _scale_body,
        grid=(NP // BN,),
        in_specs=[
            pl.BlockSpec((BN, D_IN), lambda i: (i, 0)),
            pl.BlockSpec((R, BN), lambda i: (0, i)),
            pl.BlockSpec((R, BN), lambda i: (0, i)),
        ],
        out_specs=[
            pl.BlockSpec((R, BN, D_IN), lambda i: (0, i, 0)),
            pl.BlockSpec((R, BN), lambda i: (0, i)),
        ],
        out_shape=[
            jax.ShapeDtypeStruct((R, NP, D_IN), jnp.float32),
            jax.ShapeDtypeStruct((R, NP), jnp.float32),
        ],
    )(x, deg_s2, deg_d2)


def _aggregate(kd, ks, xs):
    @functools.partial(
        pl.kernel,
        out_type=jax.ShapeDtypeStruct((NK, D_IN), jnp.float32),
        mesh=_mesh(),
        compiler_params=pltpu.CompilerParams(needs_layout_passes=False),
        scratch_types=[
            pltpu.VMEM((CHC,), jnp.int32),
            pltpu.VMEM((CHC,), jnp.int32),
            pltpu.VMEM((MAXM,), jnp.int32),
            pltpu.VMEM((MAXM,), jnp.int32),
            pltpu.VMEM((BATCH,), jnp.int32),
            pltpu.VMEM((BATCH,), jnp.int32),
            pltpu.VMEM((BATCH, D_IN), jnp.float32),
            pltpu.VMEM((BATCH,), jnp.int32),
            pltpu.VMEM((BATCH,), jnp.int32),
            pltpu.VMEM((BATCH, D_IN), jnp.float32),
            pltpu.VMEM((ZR, D_IN), jnp.float32),
            pltpu.VMEM_SHARED((CK + 8, D_IN), jnp.float32),
            pltpu.SemaphoreType.DMA,
            pltpu.SemaphoreType.DMA,
        ],
    )
    def k(kd_hbm, ks_hbm, xs_hbm, pre_hbm, kdb, ksb, didx, gidx,
          d0, g0, r0, d1, g1, r1, zbuf, acc, ssem, gsem):
        cid = lax.axis_index("c")
        sid = lax.axis_index("s")
        ebase = sid * EPT_C
        zero16 = jnp.zeros((16,), jnp.float32)

        def zb(i, _):
            row = i // (D_IN // 16)
            col = i % (D_IN // 16)
            zbuf[row, pl.ds(col * 16, 16)] = zero16
            return 0

        lax.fori_loop(0, ZR * (D_IN // 16), zb, 0)

        def fire_one(b, f):
            def do(dd, gg, rb, od, og, orb):
                @pl.when(f >= 2)
                def _():
                    pltpu.make_async_copy(rb, acc.at[dd], ssem).wait()

                def cp(kk, _):
                    dd[pl.ds(kk * 16, 16)] = didx[pl.ds(b * BATCH + kk * 16, 16)]
                    gg[pl.ds(kk * 16, 16)] = gidx[pl.ds(b * BATCH + kk * 16, 16)]
                    return 0

                lax.fori_loop(0, BATCH // 16, cp, 0)
                pltpu.async_copy(xs_hbm.at[gg], rb, gsem)

                @pl.when(f >= 1)
                def _():
                    pltpu.make_async_copy(xs_hbm.at[og], orb, gsem).wait()
                    pltpu.async_copy(orb, acc.at[od], ssem, add=True)

            @pl.when(f % 2 == 0)
            def _():
                do(d0, g0, r0, d1, g1, r1)

            @pl.when(f % 2 == 1)
            def _():
                do(d1, g1, r1, d0, g0, r0)

        def chunk_body(cc, _):
            c = cc * NC + cid
            lo = c * CK

            def zc(z, _):
                pltpu.sync_copy(zbuf, acc.at[pl.ds(sid * SLICE + z * ZR, ZR)])
                return 0

            lax.fori_loop(0, SLICE // ZR, zc, 0)
            plsc.subcore_barrier()

            def scan_blk(blk, carry):
                rem, fires = carry
                eb = ebase + blk * CHC
                pltpu.sync_copy(kd_hbm.at[pl.ds(eb, CHC)], kdb)
                pltpu.sync_copy(ks_hbm.at[pl.ds(eb, CHC)], ksb)

                def sc_in(i, cnt):
                    kdv = kdb[pl.ds(i * 16, 16)]
                    ksv = ksb[pl.ds(i * 16, 16)]
                    m = (kdv >= lo) & (kdv < lo + CK)
                    plsc.store_compressed(didx.at[pl.ds(cnt, 16)], kdv - lo, mask=m)
                    plsc.store_compressed(gidx.at[pl.ds(cnt, 16)], ksv, mask=m)
                    return cnt + plsc.all_reduce_population_count(m)[0]

                cnt = lax.fori_loop(0, CHC // 16, sc_in, rem)
                nfull = cnt // BATCH

                def fb(b, f):
                    fire_one(b, f)
                    return f + 1

                fires = lax.fori_loop(0, nfull, fb, fires)
                newrem = cnt - nfull * BATCH

                def mv(kk, _):
                    @pl.when(kk * 16 < newrem)
                    def _():
                        didx[pl.ds(kk * 16, 16)] = didx[
                            pl.ds(nfull * BATCH + kk * 16, 16)
                        ]
                        gidx[pl.ds(kk * 16, 16)] = gidx[
                            pl.ds(nfull * BATCH + kk * 16, 16)
                        ]

                    return 0

                lax.fori_loop(0, BATCH // 16, mv, 0)
                return (newrem, fires)

            rem, fires = lax.fori_loop(
                0, EPT_C // CHC, scan_blk, (jnp.int32(0), jnp.int32(0))
            )

            @pl.when(rem > 0)
            def _():
                def padb(j, _):
                    off = rem + j * 16

                    @pl.when(off < BATCH)
                    def _():
                        didx[pl.ds(off, 16)] = jnp.full((16,), CK, jnp.int32)
                        gidx[pl.ds(off, 16)] = jnp.zeros((16,), jnp.int32)

                    return 0

                lax.fori_loop(0, BATCH // 16, padb, 0)
                fire_one(0, fires)

            total = fires + (rem > 0).astype(jnp.int32)

            @pl.when(total >= 1)
            def _():
                last = (total - 1) % 2

                @pl.when(last == 0)
                def _():
                    pltpu.make_async_copy(xs_hbm.at[g0], r0, gsem).wait()
                    pltpu.async_copy(r0, acc.at[d0], ssem, add=True)

                @pl.when(last == 1)
                def _():
                    pltpu.make_async_copy(xs_hbm.at[g1], r1, gsem).wait()
                    pltpu.async_copy(r1, acc.at[d1], ssem, add=True)

            @pl.when(total >= 1)
            def _():
                pltpu.make_async_copy(r0, acc.at[d0], ssem).wait()

            @pl.when(total >= 2)
            def _():
                pltpu.make_async_copy(r0, acc.at[d0], ssem).wait()

            plsc.subcore_barrier()

            pltpu.sync_copy(
                acc.at[pl.ds(sid * SLICE, SLICE)],
                pre_hbm.at[pl.ds(lo + sid * SLICE, SLICE)],
            )
            return 0

        lax.fori_loop(0, NCH // NC, chunk_body, 0)

    return k(kd, ks, xs)


def _output_body(pre_ref, invd_ref, w3_ref, sw_ref, out_ref):
    p = pre_ref[...] * invd_ref[...][:, :, None]
    acc = jnp.dot(p[0], w3_ref[0], preferred_element_type=jnp.float32)
    for r in range(1, R):
        acc += jnp.dot(p[r], w3_ref[r], preferred_element_type=jnp.float32)
    comb = jnp.where(acc >= 0, acc, NEG * acc)
    o = jnp.dot(comb, sw_ref[...], preferred_element_type=jnp.float32)
    out_ref[...] = jnp.where(o >= 0, o, NEG * o)


def _dense_out(pre3, inv_d2, w3, sage_wt):
    BN = 512
    return pl.pallas_call(
        _output_body,
        grid=(NP // BN,),
        in_specs=[
            pl.BlockSpec((R, BN, D_IN), lambda i: (0, i, 0)),
            pl.BlockSpec((R, BN), lambda i: (0, i)),
            pl.BlockSpec((R, D_IN, D_H), lambda i: (0, 0, 0)),
            pl.BlockSpec((D_H, D_OUT), lambda i: (0, 0)),
        ],
        out_specs=pl.BlockSpec((BN, D_OUT), lambda i: (i, 0)),
        out_shape=jax.ShapeDtypeStruct((NP, D_OUT), jnp.float32),
    )(pre3, inv_d2, w3, sage_wt)


def kernel(x, edge_index, edge_rating, rating_W, sage_W):
    w3 = jnp.transpose(rating_W, (0, 2, 1))
    sage_wt = sage_W.T
    zeros_nk = jnp.zeros((NK,), jnp.float32)

    kd, ks, deg_d, deg_s = _degrees_and_keys(
        edge_index[0], edge_index[1], edge_rating, zeros_nk
    )
    degd_sum, degs_sum = _reduce_partials(
        deg_d.reshape(NW, NK), deg_s.reshape(NW, NK)
    )
    x_pad = jnp.pad(x, ((0, NP - N), (0, 0)))
    xs3, inv_d2 = _build_scaled(
        x_pad, degs_sum.reshape(R, NP), degd_sum.reshape(R, NP)
    )
    pre = _aggregate(kd, ks, xs3.reshape(NK, D_IN))
    out = _dense_out(pre.reshape(R, NP, D_IN), inv_d2, w3, sage_wt)
    return out[:N]


# speedup vs baseline: 9.0416x; 1.0960x over previous
import functools

import jax
import jax.numpy as jnp
from jax import lax
from jax.experimental import pallas as pl
from jax.experimental.pallas import tpu as pltpu
from jax.experimental.pallas import tpu_sc as plsc

R = 10
NEG = 0.1
N = 10000
NP = 10240
E = 320000
D_IN = 128
D_H = 256
D_OUT = 256
NK = R * NP

NC, NS = 2, 16
NW = NC * NS

EPT_A = E // NW
CHA = 2000

CK = 10240
NCH = NK // CK
SLICE = CK // NS
ZR = 16
EPT_C = E // NS
CHC = 2000
BATCH = 128
MAXM = CHC + BATCH + 48


def _mesh():
    return plsc.VectorSubcoreMesh(
        core_axis_name="c", subcore_axis_name="s", num_cores=NC, num_subcores=NS
    )


def _degrees_and_keys(dst, src, edge_rating, zeros_nk):
    @functools.partial(
        pl.kernel,
        out_type=(
            jax.ShapeDtypeStruct((E,), jnp.int32),
            jax.ShapeDtypeStruct((E,), jnp.int32),
            jax.ShapeDtypeStruct((NW * NK,), jnp.float32),
            jax.ShapeDtypeStruct((NW * NK,), jnp.float32),
        ),
        mesh=_mesh(),
        compiler_params=pltpu.CompilerParams(needs_layout_passes=False),
        scratch_types=[
            pltpu.VMEM((CHA,), jnp.int32),
            pltpu.VMEM((CHA,), jnp.int32),
            pltpu.VMEM((CHA,), jnp.int32),
            pltpu.VMEM((NK,), jnp.float32),
        ],
    )
    def k(dst_hbm, src_hbm, er_hbm, z_hbm, kd_hbm, ks_hbm, degd_hbm, degs_hbm,
          nbuf, rbuf, kbuf, table):
        wid = lax.axis_index("s") * NC + lax.axis_index("c")
        base = wid * EPT_A
        ones = jnp.ones((16,), jnp.float32)
        for side in range(2):
            nodes_hbm = dst_hbm if side == 0 else src_hbm
            keys_hbm = kd_hbm if side == 0 else ks_hbm
            deg_hbm = degd_hbm if side == 0 else degs_hbm
            pltpu.sync_copy(z_hbm, table)

            def chunk_body(cc, _):
                eb = base + cc * CHA
                pltpu.sync_copy(nodes_hbm.at[pl.ds(eb, CHA)], nbuf)
                pltpu.sync_copy(er_hbm.at[pl.ds(eb, CHA)], rbuf)

                def inner(i, _):
                    nd = nbuf[pl.ds(i * 16, 16)]
                    rt = rbuf[pl.ds(i * 16, 16)]
                    kk = rt * NP + nd
                    kbuf[pl.ds(i * 16, 16)] = kk
                    plsc.addupdate_scatter(table, [kk], ones)
                    return 0

                lax.fori_loop(0, CHA // 16, inner, 0)
                pltpu.sync_copy(kbuf, keys_hbm.at[pl.ds(eb, CHA)])
                return 0

            lax.fori_loop(0, EPT_A // CHA, chunk_body, 0)
            pltpu.sync_copy(table, deg_hbm.at[pl.ds(wid * NK, NK)])

    return k(dst, src, edge_rating, zeros_nk)


def _reduce_body(degd_ref, degs_ref, outd_ref, outs_ref):
    outd_ref[...] = jnp.sum(degd_ref[...], axis=0)
    outs_ref[...] = jnp.sum(degs_ref[...], axis=0)


def _reduce_partials(deg_d2, deg_s2):
    CB = 4096
    return pl.pallas_call(
        _reduce_body,
        grid=(NK // CB,),
        in_specs=[
            pl.BlockSpec((NW, CB), lambda i: (0, i)),
            pl.BlockSpec((NW, CB), lambda i: (0, i)),
        ],
        out_specs=[
            pl.BlockSpec((CB,), lambda i: (i,)),
            pl.BlockSpec((CB,), lambda i: (i,)),
        ],
        out_shape=[
            jax.ShapeDtypeStruct((NK,), jnp.float32),
            jax.ShapeDtypeStruct((NK,), jnp.float32),
        ],
    )(deg_d2, deg_s2)


def _scale_body(x_ref, degs_ref, degd_ref, xs_ref, invd_ref):
    inv_s = lax.rsqrt(jnp.maximum(degs_ref[...], 1.0))
    xs_ref[...] = x_ref[...][None, :, :] * inv_s[:, :, None]
    invd_ref[...] = lax.rsqrt(jnp.maximum(degd_ref[...], 1.0))


def _build_scaled(x, deg_s2, deg_d2):
    BN = 512
    return pl.pallas_call(
        _scale_body,
        grid=(NP // BN,),
        in_specs=[
            pl.BlockSpec((BN, D_IN), lambda i: (i, 0)),
            pl.BlockSpec((R, BN), lambda i: (0, i)),
            pl.BlockSpec((R, BN), lambda i: (0, i)),
        ],
        out_specs=[
            pl.BlockSpec((R, BN, D_IN), lambda i: (0, i, 0)),
            pl.BlockSpec((R, BN), lambda i: (0, i)),
        ],
        out_shape=[
            jax.ShapeDtypeStruct((R, NP, D_IN), jnp.float32),
            jax.ShapeDtypeStruct((R, NP), jnp.float32),
        ],
    )(x, deg_s2, deg_d2)


def _aggregate(kd, ks, xs):
    @functools.partial(
        pl.kernel,
        out_type=jax.ShapeDtypeStruct((NK, D_IN), jnp.float32),
        mesh=_mesh(),
        compiler_params=pltpu.CompilerParams(needs_layout_passes=False),
        scratch_types=[
            pltpu.VMEM((CHC,), jnp.int32),
            pltpu.VMEM((CHC,), jnp.int32),
            pltpu.VMEM((CHC,), jnp.int32),
            pltpu.VMEM((CHC,), jnp.int32),
            pltpu.VMEM((MAXM,), jnp.int32),
            pltpu.VMEM((MAXM,), jnp.int32),
            pltpu.VMEM((BATCH,), jnp.int32),
            pltpu.VMEM((BATCH,), jnp.int32),
            pltpu.VMEM((BATCH, D_IN), jnp.float32),
            pltpu.VMEM((BATCH,), jnp.int32),
            pltpu.VMEM((BATCH,), jnp.int32),
            pltpu.VMEM((BATCH, D_IN), jnp.float32),
            pltpu.VMEM((ZR, D_IN), jnp.float32),
            pltpu.VMEM_SHARED((CK + 8, D_IN), jnp.float32),
            pltpu.SemaphoreType.DMA,
            pltpu.SemaphoreType.DMA,
            pltpu.SemaphoreType.DMA,
            pltpu.SemaphoreType.DMA,
        ],
    )
    def k(kd_hbm, ks_hbm, xs_hbm, pre_hbm, kdb0, kdb1, ksb0, ksb1, didx, gidx,
          d0, g0, r0, d1, g1, r1, zbuf, acc, ssem, gsem, zsem, stsem):
        cid = lax.axis_index("c")
        sid = lax.axis_index("s")
        ebase = sid * EPT_C
        zero16 = jnp.zeros((16,), jnp.float32)
        NBLK = EPT_C // CHC
        NZ = SLICE // ZR

        def zb(i, _):
            row = i // (D_IN // 16)
            col = i % (D_IN // 16)
            zbuf[row, pl.ds(col * 16, 16)] = zero16
            return 0

        lax.fori_loop(0, ZR * (D_IN // 16), zb, 0)

        def fire_one(b, f):
            def do(dd, gg, rb, od, og, orb):
                @pl.when(f >= 2)
                def _():
                    pltpu.make_async_copy(rb, acc.at[dd], ssem).wait()

                def cp(kk, _):
                    dd[pl.ds(kk * 16, 16)] = didx[pl.ds(b * BATCH + kk * 16, 16)]
                    gg[pl.ds(kk * 16, 16)] = gidx[pl.ds(b * BATCH + kk * 16, 16)]
                    return 0

                lax.fori_loop(0, BATCH // 16, cp, 0)
                pltpu.async_copy(xs_hbm.at[gg], rb, gsem)

                @pl.when(f >= 1)
                def _():
                    pltpu.make_async_copy(xs_hbm.at[og], orb, gsem).wait()
                    pltpu.async_copy(orb, acc.at[od], ssem, add=True)

            @pl.when(f % 2 == 0)
            def _():
                do(d0, g0, r0, d1, g1, r1)

            @pl.when(f % 2 == 1)
            def _():
                do(d1, g1, r1, d0, g0, r0)

        def chunk_body(cc, _):
            c = cc * NC + cid
            lo = c * CK

            def zc(z, _):
                pltpu.async_copy(
                    zbuf, acc.at[pl.ds(sid * SLICE + z * ZR, ZR)], zsem
                )
                return 0

            lax.fori_loop(0, NZ, zc, 0)
            pltpu.async_copy(kd_hbm.at[pl.ds(ebase, CHC)], kdb0, stsem)
            pltpu.async_copy(ks_hbm.at[pl.ds(ebase, CHC)], ksb0, stsem)

            def zw(z, _):
                pltpu.make_async_copy(
                    zbuf, acc.at[pl.ds(sid * SLICE, ZR)], zsem
                ).wait()
                return 0

            lax.fori_loop(0, NZ, zw, 0)
            plsc.subcore_barrier()

            rem = jnp.int32(0)
            fires = jnp.int32(0)
            for blk in range(NBLK):
                kb, sb = (kdb0, ksb0) if blk % 2 == 0 else (kdb1, ksb1)
                nkb, nsb = (kdb1, ksb1) if blk % 2 == 0 else (kdb0, ksb0)
                pltpu.make_async_copy(
                    kd_hbm.at[pl.ds(ebase, CHC)], kb, stsem
                ).wait()
                pltpu.make_async_copy(
                    ks_hbm.at[pl.ds(ebase, CHC)], sb, stsem
                ).wait()
                if blk + 1 < NBLK:
                    eb2 = ebase + (blk + 1) * CHC
                    pltpu.async_copy(kd_hbm.at[pl.ds(eb2, CHC)], nkb, stsem)
                    pltpu.async_copy(ks_hbm.at[pl.ds(eb2, CHC)], nsb, stsem)

                def sc_in(i, cnt, kb=kb, sb=sb):
                    kdv = kb[pl.ds(i * 16, 16)]
                    ksv = sb[pl.ds(i * 16, 16)]
                    m = (kdv >= lo) & (kdv < lo + CK)
                    plsc.store_compressed(didx.at[pl.ds(cnt, 16)], kdv - lo, mask=m)
                    plsc.store_compressed(gidx.at[pl.ds(cnt, 16)], ksv, mask=m)
                    return cnt + plsc.all_reduce_population_count(m)[0]

                cnt = lax.fori_loop(0, CHC // 16, sc_in, rem)
                nfull = cnt // BATCH

                def fb(b, f):
                    fire_one(b, f)
                    return f + 1

                fires = lax.fori_loop(0, nfull, fb, fires)
                newrem = cnt - nfull * BATCH

                def mv(kk, _, nfull=nfull, newrem=newrem):
                    @pl.when(kk * 16 < newrem)
                    def _():
                        didx[pl.ds(kk * 16, 16)] = didx[
                            pl.ds(nfull * BATCH + kk * 16, 16)
                        ]
                        gidx[pl.ds(kk * 16, 16)] = gidx[
                            pl.ds(nfull * BATCH + kk * 16, 16)
                        ]

                    return 0

                lax.fori_loop(0, BATCH // 16, mv, 0)
                rem = newrem

            @pl.when(rem > 0)
            def _():
                def padb(j, _):
                    off = rem + j * 16

                    @pl.when(off < BATCH)
                    def _():
                        didx[pl.ds(off, 16)] = jnp.full((16,), CK, jnp.int32)
                        gidx[pl.ds(off, 16)] = jnp.zeros((16,), jnp.int32)

                    return 0

                lax.fori_loop(0, BATCH // 16, padb, 0)
                fire_one(0, fires)

            total = fires + (rem > 0).astype(jnp.int32)

            @pl.when(total >= 1)
            def _():
                last = (total - 1) % 2

                @pl.when(last == 0)
                def _():
                    pltpu.make_async_copy(xs_hbm.at[g0], r0, gsem).wait()
                    pltpu.async_copy(r0, acc.at[d0], ssem, add=True)

                @pl.when(last == 1)
                def _():
                    pltpu.make_async_copy(xs_hbm.at[g1], r1, gsem).wait()
                    pltpu.async_copy(r1, acc.at[d1], ssem, add=True)

            @pl.when(total >= 1)
            def _():
                pltpu.make_async_copy(r0, acc.at[d0], ssem).wait()

            @pl.when(total >= 2)
            def _():
                pltpu.make_async_copy(r0, acc.at[d0], ssem).wait()

            plsc.subcore_barrier()

            pltpu.sync_copy(
                acc.at[pl.ds(sid * SLICE, SLICE)],
                pre_hbm.at[pl.ds(lo + sid * SLICE, SLICE)],
            )
            return 0

        lax.fori_loop(0, NCH // NC, chunk_body, 0)

    return k(kd, ks, xs)


def _output_body(pre_ref, invd_ref, w3_ref, sw_ref, out_ref):
    p = pre_ref[...] * invd_ref[...][:, :, None]
    acc = jnp.dot(p[0], w3_ref[0], preferred_element_type=jnp.float32)
    for r in range(1, R):
        acc += jnp.dot(p[r], w3_ref[r], preferred_element_type=jnp.float32)
    comb = jnp.where(acc >= 0, acc, NEG * acc)
    o = jnp.dot(comb, sw_ref[...], preferred_element_type=jnp.float32)
    out_ref[...] = jnp.where(o >= 0, o, NEG * o)


def _dense_out(pre3, inv_d2, w3, sage_wt):
    BN = 512
    return pl.pallas_call(
        _output_body,
        grid=(NP // BN,),
        in_specs=[
            pl.BlockSpec((R, BN, D_IN), lambda i: (0, i, 0)),
            pl.BlockSpec((R, BN), lambda i: (0, i)),
            pl.BlockSpec((R, D_IN, D_H), lambda i: (0, 0, 0)),
            pl.BlockSpec((D_H, D_OUT), lambda i: (0, 0)),
        ],
        out_specs=pl.BlockSpec((BN, D_OUT), lambda i: (i, 0)),
        out_shape=jax.ShapeDtypeStruct((NP, D_OUT), jnp.float32),
    )(pre3, inv_d2, w3, sage_wt)


def kernel(x, edge_index, edge_rating, rating_W, sage_W):
    w3 = jnp.transpose(rating_W, (0, 2, 1))
    sage_wt = sage_W.T
    zeros_nk = jnp.zeros((NK,), jnp.float32)

    kd, ks, deg_d, deg_s = _degrees_and_keys(
        edge_index[0], edge_index[1], edge_rating, zeros_nk
    )
    degd_sum, degs_sum = _reduce_partials(
        deg_d.reshape(NW, NK), deg_s.reshape(NW, NK)
    )
    x_pad = jnp.pad(x, ((0, NP - N), (0, 0)))
    xs3, inv_d2 = _build_scaled(
        x_pad, degs_sum.reshape(R, NP), degd_sum.reshape(R, NP)
    )
    pre = _aggregate(kd, ks, xs3.reshape(NK, D_IN))
    out = _dense_out(pre.reshape(R, NP, D_IN), inv_d2, w3, sage_wt)
    return out[:N]


# speedup vs baseline: 11.6079x; 1.2838x over previous
import functools

import jax
import jax.numpy as jnp
from jax import lax
from jax.experimental import pallas as pl
from jax.experimental.pallas import tpu as pltpu
from jax.experimental.pallas import tpu_sc as plsc

R = 10
NEG = 0.1
N = 10000
NP = 10240
E = 320000
D_IN = 128
D_H = 256
D_OUT = 256
NK = R * NP

NC, NS = 2, 16
NW = NC * NS

EPT_A = E // NW
CHA = 2000

CK = 10240
NCH = NK // CK
SLICE = CK // NS
ZR = 16
EPT_C = E // NS
CHC = 2000
BATCH = 64
NSLOT = 4
RLAG = 3
MAXM = CHC + BATCH + 48


def _mesh():
    return plsc.VectorSubcoreMesh(
        core_axis_name="c", subcore_axis_name="s", num_cores=NC, num_subcores=NS
    )


def _degrees_and_keys(dst, src, edge_rating, zeros_nk):
    @functools.partial(
        pl.kernel,
        out_type=(
            jax.ShapeDtypeStruct((E,), jnp.int32),
            jax.ShapeDtypeStruct((E,), jnp.int32),
            jax.ShapeDtypeStruct((NW * NK,), jnp.float32),
            jax.ShapeDtypeStruct((NW * NK,), jnp.float32),
        ),
        mesh=_mesh(),
        compiler_params=pltpu.CompilerParams(needs_layout_passes=False),
        scratch_types=[
            pltpu.VMEM((CHA,), jnp.int32),
            pltpu.VMEM((CHA,), jnp.int32),
            pltpu.VMEM((CHA,), jnp.int32),
            pltpu.VMEM((NK,), jnp.float32),
        ],
    )
    def k(dst_hbm, src_hbm, er_hbm, z_hbm, kd_hbm, ks_hbm, degd_hbm, degs_hbm,
          nbuf, rbuf, kbuf, table):
        wid = lax.axis_index("s") * NC + lax.axis_index("c")
        base = wid * EPT_A
        ones = jnp.ones((16,), jnp.float32)
        for side in range(2):
            nodes_hbm = dst_hbm if side == 0 else src_hbm
            keys_hbm = kd_hbm if side == 0 else ks_hbm
            deg_hbm = degd_hbm if side == 0 else degs_hbm
            pltpu.sync_copy(z_hbm, table)

            def chunk_body(cc, _):
                eb = base + cc * CHA
                pltpu.sync_copy(nodes_hbm.at[pl.ds(eb, CHA)], nbuf)
                pltpu.sync_copy(er_hbm.at[pl.ds(eb, CHA)], rbuf)

                def inner(i, _):
                    nd = nbuf[pl.ds(i * 16, 16)]
                    rt = rbuf[pl.ds(i * 16, 16)]
                    kk = rt * NP + nd
                    kbuf[pl.ds(i * 16, 16)] = kk
                    plsc.addupdate_scatter(table, [kk], ones)
                    return 0

                lax.fori_loop(0, CHA // 16, inner, 0)
                pltpu.sync_copy(kbuf, keys_hbm.at[pl.ds(eb, CHA)])
                return 0

            lax.fori_loop(0, EPT_A // CHA, chunk_body, 0)
            pltpu.sync_copy(table, deg_hbm.at[pl.ds(wid * NK, NK)])

    return k(dst, src, edge_rating, zeros_nk)


def _reduce_body(degd_ref, degs_ref, outd_ref, outs_ref):
    outd_ref[...] = jnp.sum(degd_ref[...], axis=0)
    outs_ref[...] = jnp.sum(degs_ref[...], axis=0)


def _reduce_partials(deg_d2, deg_s2):
    CB = 4096
    return pl.pallas_call(
        _reduce_body,
        grid=(NK // CB,),
        in_specs=[
            pl.BlockSpec((NW, CB), lambda i: (0, i)),
            pl.BlockSpec((NW, CB), lambda i: (0, i)),
        ],
        out_specs=[
            pl.BlockSpec((CB,), lambda i: (i,)),
            pl.BlockSpec((CB,), lambda i: (i,)),
        ],
        out_shape=[
            jax.ShapeDtypeStruct((NK,), jnp.float32),
            jax.ShapeDtypeStruct((NK,), jnp.float32),
        ],
    )(deg_d2, deg_s2)


def _scale_body(x_ref, degs_ref, degd_ref, xs_ref, invd_ref):
    inv_s = lax.rsqrt(jnp.maximum(degs_ref[...], 1.0))
    xs_ref[...] = x_ref[...][None, :, :] * inv_s[:, :, None]
    invd_ref[...] = lax.rsqrt(jnp.maximum(degd_ref[...], 1.0))


def _build_scaled(x, deg_s2, deg_d2):
    BN = 512
    return pl.pallas_call(
        _scale_body,
        grid=(NP // BN,),
        in_specs=[
            pl.BlockSpec((BN, D_IN), lambda i: (i, 0)),
            pl.BlockSpec((R, BN), lambda i: (0, i)),
            pl.BlockSpec((R, BN), lambda i: (0, i)),
        ],
        out_specs=[
            pl.BlockSpec((R, BN, D_IN), lambda i: (0, i, 0)),
            pl.BlockSpec((R, BN), lambda i: (0, i)),
        ],
        out_shape=[
            jax.ShapeDtypeStruct((R, NP, D_IN), jnp.float32),
            jax.ShapeDtypeStruct((R, NP), jnp.float32),
        ],
    )(x, deg_s2, deg_d2)


def _aggregate(kd, ks, xs):
    @functools.partial(
        pl.kernel,
        out_type=jax.ShapeDtypeStruct((NK, D_IN), jnp.float32),
        mesh=_mesh(),
        compiler_params=pltpu.CompilerParams(needs_layout_passes=False),
        scratch_types=[
            pltpu.VMEM((CHC,), jnp.int32),
            pltpu.VMEM((CHC,), jnp.int32),
            pltpu.VMEM((CHC,), jnp.int32),
            pltpu.VMEM((CHC,), jnp.int32),
            pltpu.VMEM((MAXM,), jnp.int32),
            pltpu.VMEM((MAXM,), jnp.int32),
        ] + [
            t
            for _ in range(NSLOT)
            for t in (
                pltpu.VMEM((BATCH,), jnp.int32),
                pltpu.VMEM((BATCH,), jnp.int32),
                pltpu.VMEM((BATCH, D_IN), jnp.float32),
            )
        ] + [
            pltpu.VMEM((ZR, D_IN), jnp.float32),
            pltpu.VMEM_SHARED((CK + 8, D_IN), jnp.float32),
            pltpu.SemaphoreType.DMA,
            pltpu.SemaphoreType.DMA,
            pltpu.SemaphoreType.DMA,
            pltpu.SemaphoreType.DMA,
        ],
    )
    def k(kd_hbm, ks_hbm, xs_hbm, pre_hbm, kdb0, kdb1, ksb0, ksb1, didx, gidx,
          *rest):
        slots = [tuple(rest[3 * s:3 * s + 3]) for s in range(NSLOT)]
        zbuf, acc, ssem, gsem, zsem, stsem = rest[3 * NSLOT:]
        cid = lax.axis_index("c")
        sid = lax.axis_index("s")
        ebase = sid * EPT_C
        zero16 = jnp.zeros((16,), jnp.float32)
        NBLK = EPT_C // CHC
        NZ = SLICE // ZR

        def zb(i, _):
            row = i // (D_IN // 16)
            col = i % (D_IN // 16)
            zbuf[row, pl.ds(col * 16, 16)] = zero16
            return 0

        lax.fori_loop(0, ZR * (D_IN // 16), zb, 0)

        def fire_one(b, f):
            def do(s):
                dd, gg, rb = slots[s]
                od, og, orb = slots[(s - RLAG) % NSLOT]

                @pl.when(f >= NSLOT)
                def _():
                    pltpu.make_async_copy(rb, acc.at[dd], ssem).wait()

                def cp(kk, _):
                    dd[pl.ds(kk * 16, 16)] = didx[pl.ds(b * BATCH + kk * 16, 16)]
                    gg[pl.ds(kk * 16, 16)] = gidx[pl.ds(b * BATCH + kk * 16, 16)]
                    return 0

                lax.fori_loop(0, BATCH // 16, cp, 0)
                pltpu.async_copy(xs_hbm.at[gg], rb, gsem)

                @pl.when(f >= RLAG)
                def _():
                    pltpu.make_async_copy(xs_hbm.at[og], orb, gsem).wait()
                    pltpu.async_copy(orb, acc.at[od], ssem, add=True)

            for s in range(NSLOT):
                @pl.when(f % NSLOT == s)
                def _(s=s):
                    do(s)

        def chunk_body(cc, _):
            c = cc * NC + cid
            lo = c * CK

            def zc(z, _):
                pltpu.async_copy(
                    zbuf, acc.at[pl.ds(sid * SLICE + z * ZR, ZR)], zsem
                )
                return 0

            lax.fori_loop(0, NZ, zc, 0)
            pltpu.async_copy(kd_hbm.at[pl.ds(ebase, CHC)], kdb0, stsem)
            pltpu.async_copy(ks_hbm.at[pl.ds(ebase, CHC)], ksb0, stsem)

            def zw(z, _):
                pltpu.make_async_copy(
                    zbuf, acc.at[pl.ds(sid * SLICE, ZR)], zsem
                ).wait()
                return 0

            lax.fori_loop(0, NZ, zw, 0)
            plsc.subcore_barrier()

            rem = jnp.int32(0)
            fires = jnp.int32(0)
            for blk in range(NBLK):
                kb, sb = (kdb0, ksb0) if blk % 2 == 0 else (kdb1, ksb1)
                nkb, nsb = (kdb1, ksb1) if blk % 2 == 0 else (kdb0, ksb0)
                pltpu.make_async_copy(
                    kd_hbm.at[pl.ds(ebase, CHC)], kb, stsem
                ).wait()
                pltpu.make_async_copy(
                    ks_hbm.at[pl.ds(ebase, CHC)], sb, stsem
                ).wait()
                if blk + 1 < NBLK:
                    eb2 = ebase + (blk + 1) * CHC
                    pltpu.async_copy(kd_hbm.at[pl.ds(eb2, CHC)], nkb, stsem)
                    pltpu.async_copy(ks_hbm.at[pl.ds(eb2, CHC)], nsb, stsem)

                def sc_in(i, cnt, kb=kb, sb=sb):
                    kdv = kb[pl.ds(i * 16, 16)]
                    ksv = sb[pl.ds(i * 16, 16)]
                    m = (kdv >= lo) & (kdv < lo + CK)
                    plsc.store_compressed(didx.at[pl.ds(cnt, 16)], kdv - lo, mask=m)
                    plsc.store_compressed(gidx.at[pl.ds(cnt, 16)], ksv, mask=m)
                    return cnt + plsc.all_reduce_population_count(m)[0]

                cnt = lax.fori_loop(0, CHC // 16, sc_in, rem)
                nfull = cnt // BATCH

                def fb(b, f):
                    fire_one(b, f)
                    return f + 1

                fires = lax.fori_loop(0, nfull, fb, fires)
                newrem = cnt - nfull * BATCH

                def mv(kk, _, nfull=nfull, newrem=newrem):
                    @pl.when(kk * 16 < newrem)
                    def _():
                        didx[pl.ds(kk * 16, 16)] = didx[
                            pl.ds(nfull * BATCH + kk * 16, 16)
                        ]
                        gidx[pl.ds(kk * 16, 16)] = gidx[
                            pl.ds(nfull * BATCH + kk * 16, 16)
                        ]

                    return 0

                lax.fori_loop(0, BATCH // 16, mv, 0)
                rem = newrem

            @pl.when(rem > 0)
            def _():
                def padb(j, _):
                    off = rem + j * 16

                    @pl.when(off < BATCH)
                    def _():
                        didx[pl.ds(off, 16)] = jnp.full((16,), CK, jnp.int32)
                        gidx[pl.ds(off, 16)] = jnp.zeros((16,), jnp.int32)

                    return 0

                lax.fori_loop(0, BATCH // 16, padb, 0)
                fire_one(0, fires)

            total = fires + (rem > 0).astype(jnp.int32)

            for j in range(RLAG):
                f_ret = total - RLAG + j

                @pl.when(f_ret >= 0)
                def _(f_ret=f_ret):
                    for s in range(NSLOT):
                        @pl.when(f_ret % NSLOT == s)
                        def _(s=s):
                            od, og, orb = slots[s]
                            pltpu.make_async_copy(
                                xs_hbm.at[og], orb, gsem
                            ).wait()
                            pltpu.async_copy(orb, acc.at[od], ssem, add=True)

            d0_, g0_, r0_ = slots[0]
            for j in range(NSLOT):
                @pl.when(total >= j + 1)
                def _():
                    pltpu.make_async_copy(r0_, acc.at[d0_], ssem).wait()

            plsc.subcore_barrier()

            pltpu.sync_copy(
                acc.at[pl.ds(sid * SLICE, SLICE)],
                pre_hbm.at[pl.ds(lo + sid * SLICE, SLICE)],
            )
            return 0

        lax.fori_loop(0, NCH // NC, chunk_body, 0)

    return k(kd, ks, xs)


def _output_body(pre_ref, invd_ref, w3_ref, sw_ref, out_ref):
    p = pre_ref[...] * invd_ref[...][:, :, None]
    acc = jnp.dot(p[0], w3_ref[0], preferred_element_type=jnp.float32)
    for r in range(1, R):
        acc += jnp.dot(p[r], w3_ref[r], preferred_element_type=jnp.float32)
    comb = jnp.where(acc >= 0, acc, NEG * acc)
    o = jnp.dot(comb, sw_ref[...], preferred_element_type=jnp.float32)
    out_ref[...] = jnp.where(o >= 0, o, NEG * o)


def _dense_out(pre3, inv_d2, w3, sage_wt):
    BN = 512
    return pl.pallas_call(
        _output_body,
        grid=(NP // BN,),
        in_specs=[
            pl.BlockSpec((R, BN, D_IN), lambda i: (0, i, 0)),
            pl.BlockSpec((R, BN), lambda i: (0, i)),
            pl.BlockSpec((R, D_IN, D_H), lambda i: (0, 0, 0)),
            pl.BlockSpec((D_H, D_OUT), lambda i: (0, 0)),
        ],
        out_specs=pl.BlockSpec((BN, D_OUT), lambda i: (i, 0)),
        out_shape=jax.ShapeDtypeStruct((NP, D_OUT), jnp.float32),
    )(pre3, inv_d2, w3, sage_wt)


def kernel(x, edge_index, edge_rating, rating_W, sage_W):
    w3 = jnp.transpose(rating_W, (0, 2, 1))
    sage_wt = sage_W.T
    zeros_nk = jnp.zeros((NK,), jnp.float32)

    kd, ks, deg_d, deg_s = _degrees_and_keys(
        edge_index[0], edge_index[1], edge_rating, zeros_nk
    )
    degd_sum, degs_sum = _reduce_partials(
        deg_d.reshape(NW, NK), deg_s.reshape(NW, NK)
    )
    x_pad = jnp.pad(x, ((0, NP - N), (0, 0)))
    xs3, inv_d2 = _build_scaled(
        x_pad, degs_sum.reshape(R, NP), degd_sum.reshape(R, NP)
    )
    pre = _aggregate(kd, ks, xs3.reshape(NK, D_IN))
    out = _dense_out(pre.reshape(R, NP, D_IN), inv_d2, w3, sage_wt)
    return out[:N]


# speedup vs baseline: 13.6413x; 1.1752x over previous
import functools

import jax
import jax.numpy as jnp
from jax import lax
from jax.experimental import pallas as pl
from jax.experimental.pallas import tpu as pltpu
from jax.experimental.pallas import tpu_sc as plsc

R = 10
NEG = 0.1
N = 10000
NP = 10240
E = 320000
D_IN = 128
D_H = 256
D_OUT = 256
NK = R * NP

NC, NS = 2, 16
NW = NC * NS

EPT_A = E // NW
CHA = 2000

CK = 10240
NCH = NK // CK
SLICE = CK // NS
ZR = 8
EPT_C = E // NS
CHC = 2000
BATCH = 32
NSLOT = 8
RLAG = 6
MAXM = CHC + BATCH + 16


def _mesh():
    return plsc.VectorSubcoreMesh(
        core_axis_name="c", subcore_axis_name="s", num_cores=NC, num_subcores=NS
    )


def _degrees_and_keys(dst, src, edge_rating, zeros_nk):
    @functools.partial(
        pl.kernel,
        out_type=(
            jax.ShapeDtypeStruct((E,), jnp.int32),
            jax.ShapeDtypeStruct((E,), jnp.int32),
            jax.ShapeDtypeStruct((NW * NK,), jnp.float32),
            jax.ShapeDtypeStruct((NW * NK,), jnp.float32),
        ),
        mesh=_mesh(),
        compiler_params=pltpu.CompilerParams(needs_layout_passes=False),
        scratch_types=[
            pltpu.VMEM((CHA,), jnp.int32),
            pltpu.VMEM((CHA,), jnp.int32),
            pltpu.VMEM((CHA,), jnp.int32),
            pltpu.VMEM((NK,), jnp.float32),
        ],
    )
    def k(dst_hbm, src_hbm, er_hbm, z_hbm, kd_hbm, ks_hbm, degd_hbm, degs_hbm,
          nbuf, rbuf, kbuf, table):
        wid = lax.axis_index("s") * NC + lax.axis_index("c")
        base = wid * EPT_A
        ones = jnp.ones((16,), jnp.float32)
        for side in range(2):
            nodes_hbm = dst_hbm if side == 0 else src_hbm
            keys_hbm = kd_hbm if side == 0 else ks_hbm
            deg_hbm = degd_hbm if side == 0 else degs_hbm
            pltpu.sync_copy(z_hbm, table)

            def chunk_body(cc, _):
                eb = base + cc * CHA
                pltpu.sync_copy(nodes_hbm.at[pl.ds(eb, CHA)], nbuf)
                pltpu.sync_copy(er_hbm.at[pl.ds(eb, CHA)], rbuf)

                def inner(i, _):
                    nd = nbuf[pl.ds(i * 16, 16)]
                    rt = rbuf[pl.ds(i * 16, 16)]
                    kk = rt * NP + nd
                    kbuf[pl.ds(i * 16, 16)] = kk
                    plsc.addupdate_scatter(table, [kk], ones)
                    return 0

                lax.fori_loop(0, CHA // 16, inner, 0)
                pltpu.sync_copy(kbuf, keys_hbm.at[pl.ds(eb, CHA)])
                return 0

            lax.fori_loop(0, EPT_A // CHA, chunk_body, 0)
            pltpu.sync_copy(table, deg_hbm.at[pl.ds(wid * NK, NK)])

    return k(dst, src, edge_rating, zeros_nk)


def _reduce_body(degd_ref, degs_ref, outd_ref, outs_ref):
    outd_ref[...] = jnp.sum(degd_ref[...], axis=0)
    outs_ref[...] = jnp.sum(degs_ref[...], axis=0)


def _reduce_partials(deg_d2, deg_s2):
    CB = 4096
    return pl.pallas_call(
        _reduce_body,
        grid=(NK // CB,),
        in_specs=[
            pl.BlockSpec((NW, CB), lambda i: (0, i)),
            pl.BlockSpec((NW, CB), lambda i: (0, i)),
        ],
        out_specs=[
            pl.BlockSpec((CB,), lambda i: (i,)),
            pl.BlockSpec((CB,), lambda i: (i,)),
        ],
        out_shape=[
            jax.ShapeDtypeStruct((NK,), jnp.float32),
            jax.ShapeDtypeStruct((NK,), jnp.float32),
        ],
    )(deg_d2, deg_s2)


def _scale_body(x_ref, degs_ref, degd_ref, xs_ref, invd_ref):
    inv_s = lax.rsqrt(jnp.maximum(degs_ref[...], 1.0))
    xs_ref[...] = x_ref[...][None, :, :] * inv_s[:, :, None]
    invd_ref[...] = lax.rsqrt(jnp.maximum(degd_ref[...], 1.0))


def _build_scaled(x, deg_s2, deg_d2):
    BN = 512
    return pl.pallas_call(
        _scale_body,
        grid=(NP // BN,),
        in_specs=[
            pl.BlockSpec((BN, D_IN), lambda i: (i, 0)),
            pl.BlockSpec((R, BN), lambda i: (0, i)),
            pl.BlockSpec((R, BN), lambda i: (0, i)),
        ],
        out_specs=[
            pl.BlockSpec((R, BN, D_IN), lambda i: (0, i, 0)),
            pl.BlockSpec((R, BN), lambda i: (0, i)),
        ],
        out_shape=[
            jax.ShapeDtypeStruct((R, NP, D_IN), jnp.float32),
            jax.ShapeDtypeStruct((R, NP), jnp.float32),
        ],
    )(x, deg_s2, deg_d2)


def _aggregate(kd, ks, xs):
    @functools.partial(
        pl.kernel,
        out_type=jax.ShapeDtypeStruct((NK, D_IN), jnp.float32),
        mesh=_mesh(),
        compiler_params=pltpu.CompilerParams(needs_layout_passes=False),
        scratch_types=[
            pltpu.VMEM((CHC,), jnp.int32),
            pltpu.VMEM((CHC,), jnp.int32),
            pltpu.VMEM((CHC,), jnp.int32),
            pltpu.VMEM((CHC,), jnp.int32),
            pltpu.VMEM((MAXM,), jnp.int32),
            pltpu.VMEM((MAXM,), jnp.int32),
        ] + [
            t
            for _ in range(NSLOT)
            for t in (
                pltpu.VMEM((BATCH,), jnp.int32),
                pltpu.VMEM((BATCH,), jnp.int32),
                pltpu.VMEM((BATCH, D_IN), jnp.float32),
            )
        ] + [
            pltpu.VMEM((ZR, D_IN), jnp.float32),
            pltpu.VMEM_SHARED((CK + 8, D_IN), jnp.float32),
            pltpu.SemaphoreType.DMA,
            pltpu.SemaphoreType.DMA,
            pltpu.SemaphoreType.DMA,
            pltpu.SemaphoreType.DMA,
        ],
    )
    def k(kd_hbm, ks_hbm, xs_hbm, pre_hbm, kdb0, kdb1, ksb0, ksb1, didx, gidx,
          *rest):
        slots = [tuple(rest[3 * s:3 * s + 3]) for s in range(NSLOT)]
        zbuf, acc, ssem, gsem, zsem, stsem = rest[3 * NSLOT:]
        cid = lax.axis_index("c")
        sid = lax.axis_index("s")
        ebase = sid * EPT_C
        zero16 = jnp.zeros((16,), jnp.float32)
        NBLK = EPT_C // CHC
        NZ = SLICE // ZR

        def zb(i, _):
            row = i // (D_IN // 16)
            col = i % (D_IN // 16)
            zbuf[row, pl.ds(col * 16, 16)] = zero16
            return 0

        lax.fori_loop(0, ZR * (D_IN // 16), zb, 0)

        def fire_one(b, f):
            def do(s):
                dd, gg, rb = slots[s]
                od, og, orb = slots[(s - RLAG) % NSLOT]

                @pl.when(f >= NSLOT)
                def _():
                    pltpu.make_async_copy(rb, acc.at[dd], ssem).wait()

                def cp(kk, _):
                    dd[pl.ds(kk * 16, 16)] = didx[pl.ds(b * BATCH + kk * 16, 16)]
                    gg[pl.ds(kk * 16, 16)] = gidx[pl.ds(b * BATCH + kk * 16, 16)]
                    return 0

                lax.fori_loop(0, BATCH // 16, cp, 0)
                pltpu.async_copy(xs_hbm.at[gg], rb, gsem)

                @pl.when(f >= RLAG)
                def _():
                    pltpu.make_async_copy(xs_hbm.at[og], orb, gsem).wait()
                    pltpu.async_copy(orb, acc.at[od], ssem, add=True)

            for s in range(NSLOT):
                @pl.when(f % NSLOT == s)
                def _(s=s):
                    do(s)

        def chunk_body(cc, _):
            c = cc * NC + cid
            lo = c * CK

            def zc(z, _):
                pltpu.async_copy(
                    zbuf, acc.at[pl.ds(sid * SLICE + z * ZR, ZR)], zsem
                )
                return 0

            lax.fori_loop(0, NZ, zc, 0)
            pltpu.async_copy(kd_hbm.at[pl.ds(ebase, CHC)], kdb0, stsem)
            pltpu.async_copy(ks_hbm.at[pl.ds(ebase, CHC)], ksb0, stsem)

            def zw(z, _):
                pltpu.make_async_copy(
                    zbuf, acc.at[pl.ds(sid * SLICE, ZR)], zsem
                ).wait()
                return 0

            lax.fori_loop(0, NZ, zw, 0)
            plsc.subcore_barrier()

            rem = jnp.int32(0)
            fires = jnp.int32(0)
            for blk in range(NBLK):
                kb, sb = (kdb0, ksb0) if blk % 2 == 0 else (kdb1, ksb1)
                nkb, nsb = (kdb1, ksb1) if blk % 2 == 0 else (kdb0, ksb0)
                pltpu.make_async_copy(
                    kd_hbm.at[pl.ds(ebase, CHC)], kb, stsem
                ).wait()
                pltpu.make_async_copy(
                    ks_hbm.at[pl.ds(ebase, CHC)], sb, stsem
                ).wait()
                if blk + 1 < NBLK:
                    eb2 = ebase + (blk + 1) * CHC
                    pltpu.async_copy(kd_hbm.at[pl.ds(eb2, CHC)], nkb, stsem)
                    pltpu.async_copy(ks_hbm.at[pl.ds(eb2, CHC)], nsb, stsem)

                def sc_in(i, cnt, kb=kb, sb=sb):
                    kdv = kb[pl.ds(i * 16, 16)]
                    ksv = sb[pl.ds(i * 16, 16)]
                    m = (kdv >= lo) & (kdv < lo + CK)
                    plsc.store_compressed(didx.at[pl.ds(cnt, 16)], kdv - lo, mask=m)
                    plsc.store_compressed(gidx.at[pl.ds(cnt, 16)], ksv, mask=m)
                    return cnt + plsc.all_reduce_population_count(m)[0]

                cnt = lax.fori_loop(0, CHC // 16, sc_in, rem)
                nfull = cnt // BATCH

                def fb(b, f):
                    fire_one(b, f)
                    return f + 1

                fires = lax.fori_loop(0, nfull, fb, fires)
                newrem = cnt - nfull * BATCH

                def mv(kk, _, nfull=nfull, newrem=newrem):
                    @pl.when(kk * 16 < newrem)
                    def _():
                        didx[pl.ds(kk * 16, 16)] = didx[
                            pl.ds(nfull * BATCH + kk * 16, 16)
                        ]
                        gidx[pl.ds(kk * 16, 16)] = gidx[
                            pl.ds(nfull * BATCH + kk * 16, 16)
                        ]

                    return 0

                lax.fori_loop(0, BATCH // 16, mv, 0)
                rem = newrem

            @pl.when(rem > 0)
            def _():
                def padb(j, _):
                    off = rem + j * 16

                    @pl.when(off < BATCH)
                    def _():
                        didx[pl.ds(off, 16)] = jnp.full((16,), CK, jnp.int32)
                        gidx[pl.ds(off, 16)] = jnp.zeros((16,), jnp.int32)

                    return 0

                lax.fori_loop(0, BATCH // 16, padb, 0)
                fire_one(0, fires)

            total = fires + (rem > 0).astype(jnp.int32)

            for j in range(RLAG):
                f_ret = total - RLAG + j

                @pl.when(f_ret >= 0)
                def _(f_ret=f_ret):
                    for s in range(NSLOT):
                        @pl.when(f_ret % NSLOT == s)
                        def _(s=s):
                            od, og, orb = slots[s]
                            pltpu.make_async_copy(
                                xs_hbm.at[og], orb, gsem
                            ).wait()
                            pltpu.async_copy(orb, acc.at[od], ssem, add=True)

            d0_, g0_, r0_ = slots[0]
            for j in range(NSLOT):
                @pl.when(total >= j + 1)
                def _():
                    pltpu.make_async_copy(r0_, acc.at[d0_], ssem).wait()

            plsc.subcore_barrier()

            pltpu.sync_copy(
                acc.at[pl.ds(sid * SLICE, SLICE)],
                pre_hbm.at[pl.ds(lo + sid * SLICE, SLICE)],
            )
            return 0

        lax.fori_loop(0, NCH // NC, chunk_body, 0)

    return k(kd, ks, xs)


def _output_body(pre_ref, invd_ref, w3_ref, sw_ref, out_ref):
    p = pre_ref[...] * invd_ref[...][:, :, None]
    acc = jnp.dot(p[0], w3_ref[0], preferred_element_type=jnp.float32)
    for r in range(1, R):
        acc += jnp.dot(p[r], w3_ref[r], preferred_element_type=jnp.float32)
    comb = jnp.where(acc >= 0, acc, NEG * acc)
    o = jnp.dot(comb, sw_ref[...], preferred_element_type=jnp.float32)
    out_ref[...] = jnp.where(o >= 0, o, NEG * o)


def _dense_out(pre3, inv_d2, w3, sage_wt):
    BN = 512
    return pl.pallas_call(
        _output_body,
        grid=(NP // BN,),
        in_specs=[
            pl.BlockSpec((R, BN, D_IN), lambda i: (0, i, 0)),
            pl.BlockSpec((R, BN), lambda i: (0, i)),
            pl.BlockSpec((R, D_IN, D_H), lambda i: (0, 0, 0)),
            pl.BlockSpec((D_H, D_OUT), lambda i: (0, 0)),
        ],
        out_specs=pl.BlockSpec((BN, D_OUT), lambda i: (i, 0)),
        out_shape=jax.ShapeDtypeStruct((NP, D_OUT), jnp.float32),
    )(pre3, inv_d2, w3, sage_wt)


def kernel(x, edge_index, edge_rating, rating_W, sage_W):
    w3 = jnp.transpose(rating_W, (0, 2, 1))
    sage_wt = sage_W.T
    zeros_nk = jnp.zeros((NK,), jnp.float32)

    kd, ks, deg_d, deg_s = _degrees_and_keys(
        edge_index[0], edge_index[1], edge_rating, zeros_nk
    )
    degd_sum, degs_sum = _reduce_partials(
        deg_d.reshape(NW, NK), deg_s.reshape(NW, NK)
    )
    x_pad = jnp.pad(x, ((0, NP - N), (0, 0)))
    xs3, inv_d2 = _build_scaled(
        x_pad, degs_sum.reshape(R, NP), degd_sum.reshape(R, NP)
    )
    pre = _aggregate(kd, ks, xs3.reshape(NK, D_IN))
    out = _dense_out(pre.reshape(R, NP, D_IN), inv_d2, w3, sage_wt)
    return out[:N]


# speedup vs baseline: 14.6236x; 1.0720x over previous
import functools

import jax
import jax.numpy as jnp
from jax import lax
from jax.experimental import pallas as pl
from jax.experimental.pallas import tpu as pltpu
from jax.experimental.pallas import tpu_sc as plsc

R = 10
NEG = 0.1
N = 10000
NP = 10240
E = 320000
D_IN = 128
D_H = 256
D_OUT = 256
NK = R * NP

NC, NS = 2, 16
NW = NC * NS

EPT_A = E // NW
CHA = 2000

CK = 10240
NCH = NK // CK
SLICE = CK // NS
ZR = 8
EPT_C = E // NS
CHC = 2000
BATCH = 32
NSLOT = 8
RLAG = 6
MAXM = CHC + BATCH + 16


def _mesh():
    return plsc.VectorSubcoreMesh(
        core_axis_name="c", subcore_axis_name="s", num_cores=NC, num_subcores=NS
    )


def _degrees_and_keys(dst, src, edge_rating, zeros_nk):
    @functools.partial(
        pl.kernel,
        out_type=(
            jax.ShapeDtypeStruct((E,), jnp.int32),
            jax.ShapeDtypeStruct((E,), jnp.int32),
            jax.ShapeDtypeStruct((NW * NK,), jnp.float32),
            jax.ShapeDtypeStruct((NW * NK,), jnp.float32),
        ),
        mesh=_mesh(),
        compiler_params=pltpu.CompilerParams(needs_layout_passes=False),
        scratch_types=[
            pltpu.VMEM((CHA,), jnp.int32),
            pltpu.VMEM((CHA,), jnp.int32),
            pltpu.VMEM((CHA,), jnp.int32),
            pltpu.VMEM((NK,), jnp.float32),
        ],
    )
    def k(dst_hbm, src_hbm, er_hbm, z_hbm, kd_hbm, ks_hbm, degd_hbm, degs_hbm,
          nbuf, rbuf, kbuf, table):
        wid = lax.axis_index("s") * NC + lax.axis_index("c")
        base = wid * EPT_A
        ones = jnp.ones((16,), jnp.float32)
        for side in range(2):
            nodes_hbm = dst_hbm if side == 0 else src_hbm
            keys_hbm = kd_hbm if side == 0 else ks_hbm
            deg_hbm = degd_hbm if side == 0 else degs_hbm
            pltpu.sync_copy(z_hbm, table)

            def chunk_body(cc, _):
                eb = base + cc * CHA
                pltpu.sync_copy(nodes_hbm.at[pl.ds(eb, CHA)], nbuf)
                pltpu.sync_copy(er_hbm.at[pl.ds(eb, CHA)], rbuf)

                def inner(i, _):
                    nd = nbuf[pl.ds(i * 16, 16)]
                    rt = rbuf[pl.ds(i * 16, 16)]
                    kk = rt * NP + nd
                    kbuf[pl.ds(i * 16, 16)] = kk
                    plsc.addupdate_scatter(table, [kk], ones)
                    return 0

                lax.fori_loop(0, CHA // 16, inner, 0)
                pltpu.sync_copy(kbuf, keys_hbm.at[pl.ds(eb, CHA)])
                return 0

            lax.fori_loop(0, EPT_A // CHA, chunk_body, 0)
            pltpu.sync_copy(table, deg_hbm.at[pl.ds(wid * NK, NK)])

    return k(dst, src, edge_rating, zeros_nk)


def _reduce_body(degd_ref, degs_ref, outd_ref, outs_ref):
    outd_ref[...] = jnp.sum(degd_ref[...], axis=0)
    outs_ref[...] = jnp.sum(degs_ref[...], axis=0)


def _reduce_partials(deg_d2, deg_s2):
    CB = 4096
    return pl.pallas_call(
        _reduce_body,
        grid=(NK // CB,),
        in_specs=[
            pl.BlockSpec((NW, CB), lambda i: (0, i)),
            pl.BlockSpec((NW, CB), lambda i: (0, i)),
        ],
        out_specs=[
            pl.BlockSpec((CB,), lambda i: (i,)),
            pl.BlockSpec((CB,), lambda i: (i,)),
        ],
        out_shape=[
            jax.ShapeDtypeStruct((NK,), jnp.float32),
            jax.ShapeDtypeStruct((NK,), jnp.float32),
        ],
    )(deg_d2, deg_s2)


def _scale_body(x_ref, degs_ref, degd_ref, xs_ref, invd_ref):
    inv_s = lax.rsqrt(jnp.maximum(degs_ref[...], 1.0))
    xs_ref[...] = x_ref[...][None, :, :] * inv_s[:, :, None]
    invd_ref[...] = lax.rsqrt(jnp.maximum(degd_ref[...], 1.0))


def _build_scaled(x, deg_s2, deg_d2):
    BN = 512
    return pl.pallas_call(
        _scale_body,
        grid=(NP // BN,),
        in_specs=[
            pl.BlockSpec((BN, D_IN), lambda i: (i, 0)),
            pl.BlockSpec((R, BN), lambda i: (0, i)),
            pl.BlockSpec((R, BN), lambda i: (0, i)),
        ],
        out_specs=[
            pl.BlockSpec((R, BN, D_IN), lambda i: (0, i, 0)),
            pl.BlockSpec((R, BN), lambda i: (0, i)),
        ],
        out_shape=[
            jax.ShapeDtypeStruct((R, NP, D_IN), jnp.float32),
            jax.ShapeDtypeStruct((R, NP), jnp.float32),
        ],
    )(x, deg_s2, deg_d2)


def _aggregate(kd, ks, xs):
    @functools.partial(
        pl.kernel,
        out_type=jax.ShapeDtypeStruct((NK, D_IN), jnp.float32),
        mesh=_mesh(),
        compiler_params=pltpu.CompilerParams(needs_layout_passes=False),
        scratch_types=[
            pltpu.VMEM((CHC,), jnp.int32),
            pltpu.VMEM((CHC,), jnp.int32),
            pltpu.VMEM((CHC,), jnp.int32),
            pltpu.VMEM((CHC,), jnp.int32),
            pltpu.VMEM((MAXM,), jnp.int32),
            pltpu.VMEM((MAXM,), jnp.int32),
        ] + [
            t
            for _ in range(NSLOT)
            for t in (
                pltpu.VMEM((BATCH,), jnp.int32),
                pltpu.VMEM((BATCH,), jnp.int32),
                pltpu.VMEM((BATCH, D_IN), jnp.float32),
            )
        ] + [
            pltpu.VMEM((ZR, D_IN), jnp.float32),
            pltpu.VMEM_SHARED((CK + 8, D_IN), jnp.float32),
            pltpu.SemaphoreType.DMA,
            pltpu.SemaphoreType.DMA,
            pltpu.SemaphoreType.DMA,
            pltpu.SemaphoreType.DMA,
        ],
    )
    def k(kd_hbm, ks_hbm, xs_hbm, pre_hbm, kdb0, kdb1, ksb0, ksb1, didx, gidx,
          *rest):
        slots = [tuple(rest[3 * s:3 * s + 3]) for s in range(NSLOT)]
        zbuf, acc, ssem, gsem, zsem, stsem = rest[3 * NSLOT:]
        cid = lax.axis_index("c")
        sid = lax.axis_index("s")
        ebase = sid * EPT_C
        zero16 = jnp.zeros((16,), jnp.float32)
        NBLK = EPT_C // CHC
        NZ = SLICE // ZR

        def zb(i, _):
            row = i // (D_IN // 16)
            col = i % (D_IN // 16)
            zbuf[row, pl.ds(col * 16, 16)] = zero16
            return 0

        lax.fori_loop(0, ZR * (D_IN // 16), zb, 0)

        def fire_one(b, f):
            def do(s):
                dd, gg, rb = slots[s]
                od, og, orb = slots[(s - RLAG) % NSLOT]

                @pl.when(f >= NSLOT)
                def _():
                    pltpu.make_async_copy(rb, acc.at[dd], ssem).wait()

                for kk in range(BATCH // 16):
                    dd[pl.ds(kk * 16, 16)] = didx[pl.ds(b * BATCH + kk * 16, 16)]
                    gg[pl.ds(kk * 16, 16)] = gidx[pl.ds(b * BATCH + kk * 16, 16)]
                pltpu.async_copy(xs_hbm.at[gg], rb, gsem)

                @pl.when(f >= RLAG)
                def _():
                    pltpu.make_async_copy(xs_hbm.at[og], orb, gsem).wait()
                    pltpu.async_copy(orb, acc.at[od], ssem, add=True)

            for s in range(NSLOT):
                @pl.when(f % NSLOT == s)
                def _(s=s):
                    do(s)

        def chunk_body(cc, _):
            c = cc * NC + cid
            lo = c * CK

            def zc(z, _):
                pltpu.async_copy(
                    zbuf, acc.at[pl.ds(sid * SLICE + z * ZR, ZR)], zsem
                )
                return 0

            lax.fori_loop(0, NZ, zc, 0)
            pltpu.async_copy(kd_hbm.at[pl.ds(ebase, CHC)], kdb0, stsem)
            pltpu.async_copy(ks_hbm.at[pl.ds(ebase, CHC)], ksb0, stsem)

            def zw(z, _):
                pltpu.make_async_copy(
                    zbuf, acc.at[pl.ds(sid * SLICE, ZR)], zsem
                ).wait()
                return 0

            lax.fori_loop(0, NZ, zw, 0)
            plsc.subcore_barrier()

            rem = jnp.int32(0)
            fires = jnp.int32(0)
            for blk in range(NBLK):
                kb, sb = (kdb0, ksb0) if blk % 2 == 0 else (kdb1, ksb1)
                nkb, nsb = (kdb1, ksb1) if blk % 2 == 0 else (kdb0, ksb0)
                pltpu.make_async_copy(
                    kd_hbm.at[pl.ds(ebase, CHC)], kb, stsem
                ).wait()
                pltpu.make_async_copy(
                    ks_hbm.at[pl.ds(ebase, CHC)], sb, stsem
                ).wait()
                if blk + 1 < NBLK:
                    eb2 = ebase + (blk + 1) * CHC
                    pltpu.async_copy(kd_hbm.at[pl.ds(eb2, CHC)], nkb, stsem)
                    pltpu.async_copy(ks_hbm.at[pl.ds(eb2, CHC)], nsb, stsem)

                @plsc.parallel_loop(0, CHC // 16, unroll=4, carry=rem)
                def cnt(i, cnt, kb=kb, sb=sb):
                    kdv = kb[pl.ds(i * 16, 16)]
                    ksv = sb[pl.ds(i * 16, 16)]
                    m = (kdv >= lo) & (kdv < lo + CK)
                    plsc.store_compressed(didx.at[pl.ds(cnt, 16)], kdv - lo, mask=m)
                    plsc.store_compressed(gidx.at[pl.ds(cnt, 16)], ksv, mask=m)
                    return cnt + plsc.all_reduce_population_count(m)[0]
                nfull = cnt // BATCH

                def fb(b, f):
                    fire_one(b, f)
                    return f + 1

                fires = lax.fori_loop(0, nfull, fb, fires)
                newrem = cnt - nfull * BATCH

                def mv(kk, _, nfull=nfull, newrem=newrem):
                    @pl.when(kk * 16 < newrem)
                    def _():
                        didx[pl.ds(kk * 16, 16)] = didx[
                            pl.ds(nfull * BATCH + kk * 16, 16)
                        ]
                        gidx[pl.ds(kk * 16, 16)] = gidx[
                            pl.ds(nfull * BATCH + kk * 16, 16)
                        ]

                    return 0

                lax.fori_loop(0, BATCH // 16, mv, 0)
                rem = newrem

            @pl.when(rem > 0)
            def _():
                def padb(j, _):
                    off = rem + j * 16

                    @pl.when(off < BATCH)
                    def _():
                        didx[pl.ds(off, 16)] = jnp.full((16,), CK, jnp.int32)
                        gidx[pl.ds(off, 16)] = jnp.zeros((16,), jnp.int32)

                    return 0

                lax.fori_loop(0, BATCH // 16, padb, 0)
                fire_one(0, fires)

            total = fires + (rem > 0).astype(jnp.int32)

            for j in range(RLAG):
                f_ret = total - RLAG + j

                @pl.when(f_ret >= 0)
                def _(f_ret=f_ret):
                    for s in range(NSLOT):
                        @pl.when(f_ret % NSLOT == s)
                        def _(s=s):
                            od, og, orb = slots[s]
                            pltpu.make_async_copy(
                                xs_hbm.at[og], orb, gsem
                            ).wait()
                            pltpu.async_copy(orb, acc.at[od], ssem, add=True)

            d0_, g0_, r0_ = slots[0]
            for j in range(NSLOT):
                @pl.when(total >= j + 1)
                def _():
                    pltpu.make_async_copy(r0_, acc.at[d0_], ssem).wait()

            plsc.subcore_barrier()

            pltpu.sync_copy(
                acc.at[pl.ds(sid * SLICE, SLICE)],
                pre_hbm.at[pl.ds(lo + sid * SLICE, SLICE)],
            )
            return 0

        lax.fori_loop(0, NCH // NC, chunk_body, 0)

    return k(kd, ks, xs)


def _output_body(pre_ref, invd_ref, w3_ref, sw_ref, out_ref):
    p = pre_ref[...] * invd_ref[...][:, :, None]
    acc = jnp.dot(p[0], w3_ref[0], preferred_element_type=jnp.float32)
    for r in range(1, R):
        acc += jnp.dot(p[r], w3_ref[r], preferred_element_type=jnp.float32)
    comb = jnp.where(acc >= 0, acc, NEG * acc)
    o = jnp.dot(comb, sw_ref[...], preferred_element_type=jnp.float32)
    out_ref[...] = jnp.where(o >= 0, o, NEG * o)


def _dense_out(pre3, inv_d2, w3, sage_wt):
    BN = 512
    return pl.pallas_call(
        _output_body,
        grid=(NP // BN,),
        in_specs=[
            pl.BlockSpec((R, BN, D_IN), lambda i: (0, i, 0)),
            pl.BlockSpec((R, BN), lambda i: (0, i)),
            pl.BlockSpec((R, D_IN, D_H), lambda i: (0, 0, 0)),
            pl.BlockSpec((D_H, D_OUT), lambda i: (0, 0)),
        ],
        out_specs=pl.BlockSpec((BN, D_OUT), lambda i: (i, 0)),
        out_shape=jax.ShapeDtypeStruct((NP, D_OUT), jnp.float32),
    )(pre3, inv_d2, w3, sage_wt)


def kernel(x, edge_index, edge_rating, rating_W, sage_W):
    w3 = jnp.transpose(rating_W, (0, 2, 1))
    sage_wt = sage_W.T
    zeros_nk = jnp.zeros((NK,), jnp.float32)

    kd, ks, deg_d, deg_s = _degrees_and_keys(
        edge_index[0], edge_index[1], edge_rating, zeros_nk
    )
    degd_sum, degs_sum = _reduce_partials(
        deg_d.reshape(NW, NK), deg_s.reshape(NW, NK)
    )
    x_pad = jnp.pad(x, ((0, NP - N), (0, 0)))
    xs3, inv_d2 = _build_scaled(
        x_pad, degs_sum.reshape(R, NP), degd_sum.reshape(R, NP)
    )
    pre = _aggregate(kd, ks, xs3.reshape(NK, D_IN))
    out = _dense_out(pre.reshape(R, NP, D_IN), inv_d2, w3, sage_wt)
    return out[:N]


# speedup vs baseline: 15.3004x; 1.0463x over previous
import functools

import jax
import jax.numpy as jnp
from jax import lax
from jax.experimental import pallas as pl
from jax.experimental.pallas import tpu as pltpu
from jax.experimental.pallas import tpu_sc as plsc

R = 10
NEG = 0.1
N = 10000
NP = 10240
E = 320000
D_IN = 128
D_H = 256
D_OUT = 256
NK = R * NP

NC, NS = 2, 16
NW = NC * NS

EPT_A = E // NW
CHA = 2000

CK = 10240
NCH = NK // CK
SLICE = CK // NS
ZR = 8
EPT_C = E // NS
CHC = 2000
BATCH = 32
NSLOT = 8
RLAG = 6
MAXM = CHC + BATCH + 16


def _mesh():
    return plsc.VectorSubcoreMesh(
        core_axis_name="c", subcore_axis_name="s", num_cores=NC, num_subcores=NS
    )


def _degrees_and_keys(dst, src, edge_rating, zeros_nk):
    @functools.partial(
        pl.kernel,
        out_type=(
            jax.ShapeDtypeStruct((E,), jnp.int32),
            jax.ShapeDtypeStruct((E,), jnp.int32),
            jax.ShapeDtypeStruct((NW * NK,), jnp.float32),
            jax.ShapeDtypeStruct((NW * NK,), jnp.float32),
        ),
        mesh=_mesh(),
        compiler_params=pltpu.CompilerParams(needs_layout_passes=False),
        scratch_types=[
            pltpu.VMEM((CHA,), jnp.int32),
            pltpu.VMEM((CHA,), jnp.int32),
            pltpu.VMEM((CHA,), jnp.int32),
            pltpu.VMEM((CHA,), jnp.int32),
            pltpu.VMEM((CHA,), jnp.int32),
            pltpu.VMEM((CHA,), jnp.int32),
            pltpu.VMEM((NK,), jnp.float32),
            pltpu.SemaphoreType.DMA,
            pltpu.SemaphoreType.DMA,
        ],
    )
    def k(dst_hbm, src_hbm, er_hbm, z_hbm, kd_hbm, ks_hbm, degd_hbm, degs_hbm,
          nbuf0, nbuf1, rbuf0, rbuf1, kbuf0, kbuf1, table, stsem, wsem):
        wid = lax.axis_index("s") * NC + lax.axis_index("c")
        base = wid * EPT_A
        ones = jnp.ones((16,), jnp.float32)
        NBLK = EPT_A // CHA
        for side in range(2):
            nodes_hbm = dst_hbm if side == 0 else src_hbm
            keys_hbm = kd_hbm if side == 0 else ks_hbm
            deg_hbm = degd_hbm if side == 0 else degs_hbm
            pltpu.async_copy(nodes_hbm.at[pl.ds(base, CHA)], nbuf0, stsem)
            pltpu.async_copy(er_hbm.at[pl.ds(base, CHA)], rbuf0, stsem)
            pltpu.sync_copy(z_hbm, table)

            for cc in range(NBLK):
                eb = base + cc * CHA
                nb, rb, kb = (
                    (nbuf0, rbuf0, kbuf0) if cc % 2 == 0
                    else (nbuf1, rbuf1, kbuf1)
                )
                nnb, nrb = (
                    (nbuf1, rbuf1) if cc % 2 == 0 else (nbuf0, rbuf0)
                )
                pltpu.make_async_copy(
                    nodes_hbm.at[pl.ds(eb, CHA)], nb, stsem
                ).wait()
                pltpu.make_async_copy(
                    er_hbm.at[pl.ds(eb, CHA)], rb, stsem
                ).wait()
                if cc + 1 < NBLK:
                    eb2 = eb + CHA
                    pltpu.async_copy(nodes_hbm.at[pl.ds(eb2, CHA)], nnb, stsem)
                    pltpu.async_copy(er_hbm.at[pl.ds(eb2, CHA)], nrb, stsem)
                if cc >= 2:
                    pltpu.make_async_copy(
                        kb, keys_hbm.at[pl.ds(eb, CHA)], wsem
                    ).wait()

                @plsc.parallel_loop(0, CHA // 16, unroll=4)
                def _(i, nb=nb, rb=rb, kb=kb):
                    nd = nb[pl.ds(i * 16, 16)]
                    rt = rb[pl.ds(i * 16, 16)]
                    kk = rt * NP + nd
                    kb[pl.ds(i * 16, 16)] = kk
                    plsc.addupdate_scatter(table, [kk], ones)

                pltpu.async_copy(kb, keys_hbm.at[pl.ds(eb, CHA)], wsem)

            for _ in range(2):
                pltpu.make_async_copy(
                    kbuf0, keys_hbm.at[pl.ds(base, CHA)], wsem
                ).wait()
            pltpu.sync_copy(table, deg_hbm.at[pl.ds(wid * NK, NK)])

    return k(dst, src, edge_rating, zeros_nk)


def _reduce_body(degd_ref, degs_ref, outd_ref, outs_ref):
    outd_ref[...] = jnp.sum(degd_ref[...], axis=0)
    outs_ref[...] = jnp.sum(degs_ref[...], axis=0)


def _reduce_partials(deg_d2, deg_s2):
    CB = 4096
    return pl.pallas_call(
        _reduce_body,
        grid=(NK // CB,),
        in_specs=[
            pl.BlockSpec((NW, CB), lambda i: (0, i)),
            pl.BlockSpec((NW, CB), lambda i: (0, i)),
        ],
        out_specs=[
            pl.BlockSpec((CB,), lambda i: (i,)),
            pl.BlockSpec((CB,), lambda i: (i,)),
        ],
        out_shape=[
            jax.ShapeDtypeStruct((NK,), jnp.float32),
            jax.ShapeDtypeStruct((NK,), jnp.float32),
        ],
    )(deg_d2, deg_s2)


def _scale_body(x_ref, degs_ref, degd_ref, xs_ref, invd_ref):
    inv_s = lax.rsqrt(jnp.maximum(degs_ref[...], 1.0))
    xs_ref[...] = x_ref[...][None, :, :] * inv_s[:, :, None]
    invd_ref[...] = lax.rsqrt(jnp.maximum(degd_ref[...], 1.0))


def _build_scaled(x, deg_s2, deg_d2):
    BN = 512
    return pl.pallas_call(
        _scale_body,
        grid=(NP // BN,),
        in_specs=[
            pl.BlockSpec((BN, D_IN), lambda i: (i, 0)),
            pl.BlockSpec((R, BN), lambda i: (0, i)),
            pl.BlockSpec((R, BN), lambda i: (0, i)),
        ],
        out_specs=[
            pl.BlockSpec((R, BN, D_IN), lambda i: (0, i, 0)),
            pl.BlockSpec((R, BN), lambda i: (0, i)),
        ],
        out_shape=[
            jax.ShapeDtypeStruct((R, NP, D_IN), jnp.float32),
            jax.ShapeDtypeStruct((R, NP), jnp.float32),
        ],
    )(x, deg_s2, deg_d2)


def _aggregate(kd, ks, xs):
    @functools.partial(
        pl.kernel,
        out_type=jax.ShapeDtypeStruct((NK, D_IN), jnp.float32),
        mesh=_mesh(),
        compiler_params=pltpu.CompilerParams(needs_layout_passes=False),
        scratch_types=[
            pltpu.VMEM((CHC,), jnp.int32),
            pltpu.VMEM((CHC,), jnp.int32),
            pltpu.VMEM((CHC,), jnp.int32),
            pltpu.VMEM((CHC,), jnp.int32),
            pltpu.VMEM((MAXM,), jnp.int32),
            pltpu.VMEM((MAXM,), jnp.int32),
        ] + [
            t
            for _ in range(NSLOT)
            for t in (
                pltpu.VMEM((BATCH,), jnp.int32),
                pltpu.VMEM((BATCH,), jnp.int32),
                pltpu.VMEM((BATCH, D_IN), jnp.float32),
            )
        ] + [
            pltpu.VMEM((ZR, D_IN), jnp.float32),
            pltpu.VMEM_SHARED((CK + 8, D_IN), jnp.float32),
            pltpu.SemaphoreType.DMA,
            pltpu.SemaphoreType.DMA,
            pltpu.SemaphoreType.DMA,
            pltpu.SemaphoreType.DMA,
        ],
    )
    def k(kd_hbm, ks_hbm, xs_hbm, pre_hbm, kdb0, kdb1, ksb0, ksb1, didx, gidx,
          *rest):
        slots = [tuple(rest[3 * s:3 * s + 3]) for s in range(NSLOT)]
        zbuf, acc, ssem, gsem, zsem, stsem = rest[3 * NSLOT:]
        cid = lax.axis_index("c")
        sid = lax.axis_index("s")
        ebase = sid * EPT_C
        zero16 = jnp.zeros((16,), jnp.float32)
        NBLK = EPT_C // CHC
        NZ = SLICE // ZR

        def zb(i, _):
            row = i // (D_IN // 16)
            col = i % (D_IN // 16)
            zbuf[row, pl.ds(col * 16, 16)] = zero16
            return 0

        lax.fori_loop(0, ZR * (D_IN // 16), zb, 0)

        def fire_one(b, f):
            def do(s):
                dd, gg, rb = slots[s]
                od, og, orb = slots[(s - RLAG) % NSLOT]

                @pl.when(f >= NSLOT)
                def _():
                    pltpu.make_async_copy(rb, acc.at[dd], ssem).wait()

                for kk in range(BATCH // 16):
                    dd[pl.ds(kk * 16, 16)] = didx[pl.ds(b * BATCH + kk * 16, 16)]
                    gg[pl.ds(kk * 16, 16)] = gidx[pl.ds(b * BATCH + kk * 16, 16)]
                pltpu.async_copy(xs_hbm.at[gg], rb, gsem)

                @pl.when(f >= RLAG)
                def _():
                    pltpu.make_async_copy(xs_hbm.at[og], orb, gsem).wait()
                    pltpu.async_copy(orb, acc.at[od], ssem, add=True)

            for s in range(NSLOT):
                @pl.when(f % NSLOT == s)
                def _(s=s):
                    do(s)

        def chunk_body(cc, _):
            c = cc * NC + cid
            lo = c * CK

            def zc(z, _):
                pltpu.async_copy(
                    zbuf, acc.at[pl.ds(sid * SLICE + z * ZR, ZR)], zsem
                )
                return 0

            lax.fori_loop(0, NZ, zc, 0)
            pltpu.async_copy(kd_hbm.at[pl.ds(ebase, CHC)], kdb0, stsem)
            pltpu.async_copy(ks_hbm.at[pl.ds(ebase, CHC)], ksb0, stsem)

            def zw(z, _):
                pltpu.make_async_copy(
                    zbuf, acc.at[pl.ds(sid * SLICE, ZR)], zsem
                ).wait()
                return 0

            lax.fori_loop(0, NZ, zw, 0)
            plsc.subcore_barrier()

            rem = jnp.int32(0)
            fires = jnp.int32(0)
            for blk in range(NBLK):
                kb, sb = (kdb0, ksb0) if blk % 2 == 0 else (kdb1, ksb1)
                nkb, nsb = (kdb1, ksb1) if blk % 2 == 0 else (kdb0, ksb0)
                pltpu.make_async_copy(
                    kd_hbm.at[pl.ds(ebase, CHC)], kb, stsem
                ).wait()
                pltpu.make_async_copy(
                    ks_hbm.at[pl.ds(ebase, CHC)], sb, stsem
                ).wait()
                if blk + 1 < NBLK:
                    eb2 = ebase + (blk + 1) * CHC
                    pltpu.async_copy(kd_hbm.at[pl.ds(eb2, CHC)], nkb, stsem)
                    pltpu.async_copy(ks_hbm.at[pl.ds(eb2, CHC)], nsb, stsem)

                @plsc.parallel_loop(0, CHC // 16, unroll=4, carry=rem)
                def cnt(i, cnt, kb=kb, sb=sb):
                    kdv = kb[pl.ds(i * 16, 16)]
                    ksv = sb[pl.ds(i * 16, 16)]
                    m = (kdv >= lo) & (kdv < lo + CK)
                    plsc.store_compressed(didx.at[pl.ds(cnt, 16)], kdv - lo, mask=m)
                    plsc.store_compressed(gidx.at[pl.ds(cnt, 16)], ksv, mask=m)
                    return cnt + plsc.all_reduce_population_count(m)[0]
                nfull = cnt // BATCH

                def fb(b, f):
                    fire_one(b, f)
                    return f + 1

                fires = lax.fori_loop(0, nfull, fb, fires)
                newrem = cnt - nfull * BATCH

                def mv(kk, _, nfull=nfull, newrem=newrem):
                    @pl.when(kk * 16 < newrem)
                    def _():
                        didx[pl.ds(kk * 16, 16)] = didx[
                            pl.ds(nfull * BATCH + kk * 16, 16)
                        ]
                        gidx[pl.ds(kk * 16, 16)] = gidx[
                            pl.ds(nfull * BATCH + kk * 16, 16)
                        ]

                    return 0

                lax.fori_loop(0, BATCH // 16, mv, 0)
                rem = newrem

            @pl.when(rem > 0)
            def _():
                def padb(j, _):
                    off = rem + j * 16

                    @pl.when(off < BATCH)
                    def _():
                        didx[pl.ds(off, 16)] = jnp.full((16,), CK, jnp.int32)
                        gidx[pl.ds(off, 16)] = jnp.zeros((16,), jnp.int32)

                    return 0

                lax.fori_loop(0, BATCH // 16, padb, 0)
                fire_one(0, fires)

            total = fires + (rem > 0).astype(jnp.int32)

            for j in range(RLAG):
                f_ret = total - RLAG + j

                @pl.when(f_ret >= 0)
                def _(f_ret=f_ret):
                    for s in range(NSLOT):
                        @pl.when(f_ret % NSLOT == s)
                        def _(s=s):
                            od, og, orb = slots[s]
                            pltpu.make_async_copy(
                                xs_hbm.at[og], orb, gsem
                            ).wait()
                            pltpu.async_copy(orb, acc.at[od], ssem, add=True)

            d0_, g0_, r0_ = slots[0]
            for j in range(NSLOT):
                @pl.when(total >= j + 1)
                def _():
                    pltpu.make_async_copy(r0_, acc.at[d0_], ssem).wait()

            plsc.subcore_barrier()

            pltpu.sync_copy(
                acc.at[pl.ds(sid * SLICE, SLICE)],
                pre_hbm.at[pl.ds(lo + sid * SLICE, SLICE)],
            )
            return 0

        lax.fori_loop(0, NCH // NC, chunk_body, 0)

    return k(kd, ks, xs)


def _output_body(pre_ref, invd_ref, w3_ref, sw_ref, out_ref):
    p = pre_ref[...] * invd_ref[...][:, :, None]
    acc = jnp.dot(p[0], w3_ref[0], preferred_element_type=jnp.float32)
    for r in range(1, R):
        acc += jnp.dot(p[r], w3_ref[r], preferred_element_type=jnp.float32)
    comb = jnp.where(acc >= 0, acc, NEG * acc)
    o = jnp.dot(comb, sw_ref[...], preferred_element_type=jnp.float32)
    out_ref[...] = jnp.where(o >= 0, o, NEG * o)


def _dense_out(pre3, inv_d2, w3, sage_wt):
    BN = 512
    return pl.pallas_call(
        _output_body,
        grid=(NP // BN,),
        in_specs=[
            pl.BlockSpec((R, BN, D_IN), lambda i: (0, i, 0)),
            pl.BlockSpec((R, BN), lambda i: (0, i)),
            pl.BlockSpec((R, D_IN, D_H), lambda i: (0, 0, 0)),
            pl.BlockSpec((D_H, D_OUT), lambda i: (0, 0)),
        ],
        out_specs=pl.BlockSpec((BN, D_OUT), lambda i: (i, 0)),
        out_shape=jax.ShapeDtypeStruct((NP, D_OUT), jnp.float32),
    )(pre3, inv_d2, w3, sage_wt)


def kernel(x, edge_index, edge_rating, rating_W, sage_W):
    w3 = jnp.transpose(rating_W, (0, 2, 1))
    sage_wt = sage_W.T
    zeros_nk = jnp.zeros((NK,), jnp.float32)

    kd, ks, deg_d, deg_s = _degrees_and_keys(
        edge_index[0], edge_index[1], edge_rating, zeros_nk
    )
    degd_sum, degs_sum = _reduce_partials(
        deg_d.reshape(NW, NK), deg_s.reshape(NW, NK)
    )
    x_pad = jnp.pad(x, ((0, NP - N), (0, 0)))
    xs3, inv_d2 = _build_scaled(
        x_pad, degs_sum.reshape(R, NP), degd_sum.reshape(R, NP)
    )
    pre = _aggregate(kd, ks, xs3.reshape(NK, D_IN))
    out = _dense_out(pre.reshape(R, NP, D_IN), inv_d2, w3, sage_wt)
    return out[:N]
